# baseline (speedup 1.0000x reference)
import functools

import jax
import jax.numpy as jnp
from jax import lax
from jax.experimental import pallas as pl
from jax.experimental.pallas import tpu as pltpu
from jax.experimental.pallas import tpu_sc as plsc

N = 10000
IN_CH = 256
EMB = 128
HEADS = 4
HC = HEADS * EMB
FF = 512
NEG = 0.2
EPS = 1e-5
E = 160000

NP = 10240
EP = 172032
NC, NS, L = 2, 16, 16
TILE_A = EP // (NC * NS)
TILE_B = EP // NS
GA = 128
GS = 16
GB = 128
NBA = TILE_A // GA
NBB = TILE_B // GB
DEN_W = NP * 4
DMR, DMC = DEN_W // 128, 128


def _dyn_gather16(v, idx):
    dnums = lax.GatherDimensionNumbers(
        offset_dims=(), collapsed_slice_dims=(0,), start_index_map=(0,))
    return lax.gather(v, idx[:, None], dnums, slice_sizes=(1,),
                      mode=lax.GatherScatterMode.PROMISE_IN_BOUNDS)


def _k1_body(x_ref, wl_ref, bl_ref, wr_ref, br_ref,
             xl_ref, xr_ref, c0_ref, c1_ref, c2_ref, c3_ref):
    x = x_ref[...]
    xl = jnp.dot(x, wl_ref[...], preferred_element_type=jnp.float32) + bl_ref[...]
    xr = jnp.dot(x, wr_ref[...], preferred_element_type=jnp.float32) + br_ref[...]
    xl_ref[...] = xl
    xr_ref[...] = xr
    c0_ref[...] = xl[:, 0:128]
    c1_ref[...] = xl[:, 128:256]
    c2_ref[...] = xl[:, 256:384]
    c3_ref[...] = xl[:, 384:512]


def _k1(x, Wl, bl2, Wr, br2):
    blk = NP // 8
    return pl.pallas_call(
        _k1_body,
        grid=(8,),
        in_specs=[
            pl.BlockSpec((blk, IN_CH), lambda i: (i, 0)),
            pl.BlockSpec((IN_CH, HC), lambda i: (0, 0)),
            pl.BlockSpec((1, HC), lambda i: (0, 0)),
            pl.BlockSpec((IN_CH, HC), lambda i: (0, 0)),
            pl.BlockSpec((1, HC), lambda i: (0, 0)),
        ],
        out_specs=[
            pl.BlockSpec((blk, HC), lambda i: (i, 0)),
            pl.BlockSpec((blk, HC), lambda i: (i, 0)),
            pl.BlockSpec((blk, EMB), lambda i: (i, 0)),
            pl.BlockSpec((blk, EMB), lambda i: (i, 0)),
            pl.BlockSpec((blk, EMB), lambda i: (i, 0)),
            pl.BlockSpec((blk, EMB), lambda i: (i, 0)),
        ],
        out_shape=[
            jax.ShapeDtypeStruct((NP, HC), jnp.float32),
            jax.ShapeDtypeStruct((NP, HC), jnp.float32),
            jax.ShapeDtypeStruct((NP, EMB), jnp.float32),
            jax.ShapeDtypeStruct((NP, EMB), jnp.float32),
            jax.ShapeDtypeStruct((NP, EMB), jnp.float32),
            jax.ShapeDtypeStruct((NP, EMB), jnp.float32),
        ],
    )(x, Wl, bl2, Wr, br2)


def _pass_a(xl, xr, src, dst, att_flat):
    mesh = plsc.VectorSubcoreMesh(core_axis_name="c", subcore_axis_name="s")

    @functools.partial(
        pl.kernel,
        mesh=mesh,
        compiler_params=pltpu.CompilerParams(needs_layout_passes=False),
        out_type=[
            jax.ShapeDtypeStruct((4, EP), jnp.float32),
            jax.ShapeDtypeStruct((NC * NS * DEN_W,), jnp.float32),
        ],
        scratch_types=[
            pltpu.VMEM((128,), jnp.int32),
            pltpu.VMEM((128,), jnp.int32),
            pltpu.VMEM((GS, HC // 2), jnp.int32),
            pltpu.VMEM((GS, HC // 2), jnp.int32),
            pltpu.VMEM((GS, HC // 2), jnp.int32),
            pltpu.VMEM((GS, HC // 2), jnp.int32),
            pltpu.VMEM((4, 128), jnp.float32),
            pltpu.VMEM((HC,), jnp.float32),
            pltpu.VMEM((DEN_W,), jnp.float32),
            pltpu.SemaphoreType.DMA,
            pltpu.SemaphoreType.DMA,
            pltpu.SemaphoreType.DMA,
            pltpu.SemaphoreType.DMA,
        ],
    )
    def k(xl_hbm, xr_hbm, src_hbm, dst_hbm, att_hbm, ex_hbm, den_hbm,
          idx_s, idx_d, rl0, rl1, rr0, rr1, ex_buf, att_v, den_v,
          sl0, sl1, sr0, sr1):
        cid = lax.axis_index("c")
        sid = lax.axis_index("s")
        tid = sid * NC + cid
        ii = lax.iota(jnp.int32, L)
        zv = jnp.zeros((L,), jnp.float32)
        RL, RR = (rl0, rl1), (rr0, rr1)
        SL, SR = (sl0, sl1), (sr0, sr1)

        def zb(i, _):
            den_v[pl.ds(lax.mul(i, L), L)] = zv
            return 0
        lax.fori_loop(0, DEN_W // L, zb, 0)

        pltpu.sync_copy(att_hbm, att_v)
        att_e = [[att_v[pl.ds(h * EMB + w * 2 * L, L)] for w in range(4)]
                 for h in range(HEADS)]
        att_o = [[att_v[pl.ds(h * EMB + w * 2 * L + L, L)] for w in range(4)]
                 for h in range(HEADS)]

        ebase = lax.mul(tid, TILE_A)

        def issue(q, p):
            hl = pltpu.async_copy(
                xl_hbm.at[idx_s.at[pl.ds(q * GS, GS)]], RL[p], SL[p])
            hr = pltpu.async_copy(
                xr_hbm.at[idx_d.at[pl.ds(q * GS, GS)]], RR[p], SR[p])
            return hl, hr

        def quad(jq, _):
            base = jq * 128
            eb = ebase + base
            pltpu.sync_copy(src_hbm.at[pl.ds(eb, 128)], idx_s)
            pltpu.sync_copy(dst_hbm.at[pl.ds(eb, 128)], idx_d)
            hh = [issue(0, 0), issue(1, 1)]
            for q in range(8):
                p = q & 1
                hl, hr = hh[p]
                hl.wait()
                hr.wait()
                rl, rr = RL[p], RR[p]
                for sub in (0,):
                    col = q * GS + sub

                    def edge(g, lv):
                        gg = sub + g
                        sel = ii == g
                        for h in range(HEADS):
                            acc = jnp.zeros((L,), jnp.float32)
                            for w in range(4):
                                off = h * (EMB // 2) + w * L
                                wl = rl[gg, pl.ds(off, L)]
                                wr = rr[gg, pl.ds(off, L)]
                                le = plsc.bitcast(
                                    lax.shift_left(wl, 16), jnp.float32)
                                lo = plsc.bitcast(
                                    lax.bitwise_and(wl, -65536), jnp.float32)
                                re_ = plsc.bitcast(
                                    lax.shift_left(wr, 16), jnp.float32)
                                ro = plsc.bitcast(
                                    lax.bitwise_and(wr, -65536), jnp.float32)
                                te = le + re_
                                te = jnp.maximum(te, NEG * te)
                                acc = acc + te * att_e[h][w]
                                to = lo + ro
                                to = jnp.maximum(to, NEG * to)
                                acc = acc + to * att_o[h][w]
                            red = acc
                            for st in (8, 4, 2, 1):
                                red = red + _dyn_gather16(
                                    red, lax.bitwise_xor(ii, st))
                            lv = (lv[:h] + (jnp.where(sel, red, lv[h]),)
                                  + lv[h + 1:])
                        return lv
                    lv = lax.fori_loop(0, L, edge, (zv, zv, zv, zv))
                    dv = idx_d[pl.ds(q * GS, L)]
                    for h in range(HEADS):
                        ev = jnp.exp(lv[h])
                        ex_buf[h, pl.ds(col, L)] = ev
                        plsc.addupdate_scatter(den_v, [dv * 4 + h], ev)
                if q < 6:
                    hh[p] = issue(q + 2, p)
            for h in range(HEADS):
                pltpu.sync_copy(ex_buf.at[h],
                                ex_hbm.at[h].at[pl.ds(eb, 128)])
            return 0
        lax.fori_loop(0, TILE_A // 128, quad, 0)

        pltpu.sync_copy(den_v,
                        den_hbm.at[pl.ds(lax.mul(tid, DEN_W), DEN_W)])

    return k(xl, xr, src, dst, att_flat)


A5B = 384


def _pass_a5(dst, ex, denm):
    mesh = plsc.VectorSubcoreMesh(core_axis_name="c", subcore_axis_name="s")

    @functools.partial(
        pl.kernel,
        mesh=mesh,
        compiler_params=pltpu.CompilerParams(needs_layout_passes=False),
        out_type=jax.ShapeDtypeStruct((4, EP), jnp.float32),
        scratch_types=[
            pltpu.VMEM((A5B,), jnp.int32),
            pltpu.VMEM((4, A5B), jnp.float32),
            pltpu.VMEM((4, A5B), jnp.float32),
            pltpu.VMEM((DEN_W,), jnp.float32),
        ],
    )
    def k(dst_hbm, ex_hbm, den_hbm, al_hbm, idx_d, exb, alb, d_v):
        cid = lax.axis_index("c")
        sid = lax.axis_index("s")
        tid = sid * NC + cid
        ii = lax.iota(jnp.int32, L)
        pltpu.sync_copy(den_hbm, d_v)
        ebase = lax.mul(tid, TILE_A)

        def batch(b, _):
            eb = ebase + b * A5B
            pltpu.sync_copy(dst_hbm.at[pl.ds(eb, A5B)], idx_d)
            for h in range(HEADS):
                pltpu.sync_copy(ex_hbm.at[h].at[pl.ds(eb, A5B)],
                                exb.at[h].at[pl.ds(0, A5B)])
            for sub in range(0, A5B, L):
                dv = idx_d[pl.ds(sub, L)]
                for h in range(HEADS):
                    fi = dv * 4 + h
                    den = plsc.load_gather(d_v, [fi])
                    alb[h, pl.ds(sub, L)] = exb[h, pl.ds(sub, L)] / den
            for h in range(HEADS):
                pltpu.sync_copy(alb.at[h].at[pl.ds(0, A5B)],
                                al_hbm.at[h].at[pl.ds(eb, A5B)])
            return 0
        lax.fori_loop(0, TILE_A // A5B, batch, 0)

    return k(dst, ex, denm)


GBB = 64
BLK_B = 512


def _pass_b(src, dst2, alpha, t0, t1, t2, t3):
    mesh = plsc.VectorSubcoreMesh(core_axis_name="c", subcore_axis_name="s")

    @functools.partial(
        pl.kernel,
        mesh=mesh,
        compiler_params=pltpu.CompilerParams(needs_layout_passes=False),
        out_type=[jax.ShapeDtypeStruct((NP, EMB), jnp.float32)
                  for _ in range(4)],
        scratch_types=[
            pltpu.VMEM((BLK_B // GBB, GBB), jnp.int32),
            pltpu.VMEM((BLK_B // GBB, GBB), jnp.int32),
            pltpu.VMEM((BLK_B,), jnp.float32),
            pltpu.VMEM((GBB, EMB), jnp.float32),
            pltpu.VMEM((GBB, EMB), jnp.float32),
            pltpu.VMEM((GBB, EMB), jnp.float32),
            pltpu.VMEM((GBB, EMB), jnp.float32),
            pltpu.VMEM_SHARED((NP, EMB), jnp.float32),
            pltpu.SemaphoreType.DMA,
            pltpu.SemaphoreType.DMA,
            pltpu.SemaphoreType.DMA,
            pltpu.SemaphoreType.DMA,
            pltpu.SemaphoreType.DMA,
            pltpu.SemaphoreType.DMA,
            pltpu.SemaphoreType.DMA,
            pltpu.SemaphoreType.DMA,
        ],
    )
    def k(src2_hbm, dst2_hbm, al_hbm, t0_hbm, t1_hbm, t2_hbm, t3_hbm,
          o0_hbm, o1_hbm, o2_hbm, o3_hbm,
          idx_s2, idx_d2, alb, rb0, rb1, rb2, rb3, sh_acc,
          sg0, sg1, sg2, sg3, ss0, ss1, ss2, ss3):
        cid = lax.axis_index("c")
        sid = lax.axis_index("s")
        ii = lax.iota(jnp.int32, L)
        zv = jnp.zeros((L,), jnp.float32)
        RB = (rb0, rb1, rb2, rb3)
        SG = (sg0, sg1, sg2, sg3)
        SS = (ss0, ss1, ss2, ss3)

        nrows = NP // NS
        r0 = lax.mul(sid, nrows)
        ebase = lax.mul(sid, TILE_B)
        rbase = lax.mul(sid, TILE_B // GBB)

        def chunk_pass(tbl, obl, c):
            def zr(i, _):
                rb0[lax.shift_right_logical(i, 3),
                    pl.ds(lax.mul(lax.bitwise_and(i, 7), L), L)] = zv
                return 0
            lax.fori_loop(0, GBB * (EMB // L), zr, 0)
            for q in range(nrows // GBB):
                pltpu.sync_copy(rb0, sh_acc.at[pl.ds(r0 + q * GBB, GBB)])
            plsc.subcore_barrier()

            def issue_g(ib, p):
                return pltpu.async_copy(
                    tbl.at[idx_s2.at[ib]], RB[p], SG[p])

            def block(b, _):
                eb = ebase + b * BLK_B
                rr = rbase + b * (BLK_B // GBB)
                pltpu.sync_copy(src2_hbm.at[pl.ds(rr, BLK_B // GBB)], idx_s2)
                pltpu.sync_copy(dst2_hbm.at[pl.ds(rr, BLK_B // GBB)], idx_d2)
                pltpu.sync_copy(al_hbm.at[c].at[pl.ds(eb, BLK_B)], alb)
                hg = [issue_g(0, 0), issue_g(1, 1), None, None]
                hs = [None, None, None, None]
                for ib in range(BLK_B // GBB):
                    p = ib & 3
                    hg[p].wait()
                    rows = RB[p]
                    for sub in range(0, GBB, L):
                        av = alb[pl.ds(ib * GBB + sub, L)]

                        def edge(g, _):
                            gg = sub + g
                            bc = _dyn_gather16(av, jnp.full((L,), g, jnp.int32))
                            for j in range(EMB // L):
                                rows[gg, pl.ds(j * L, L)] = (
                                    rows[gg, pl.ds(j * L, L)] * bc)
                            return 0
                        lax.fori_loop(0, L, edge, 0)
                    hs[p] = pltpu.async_copy(
                        rows, sh_acc.at[idx_d2.at[ib]], SS[p], add=True)
                    if ib < BLK_B // GBB - 2:
                        pn = (ib + 2) & 3
                        if hs[pn] is not None:
                            hs[pn].wait()
                        hg[pn] = issue_g(ib + 2, pn)
                for p in range(4):
                    hs[p].wait()
                return 0
            lax.fori_loop(0, TILE_B // BLK_B, block, 0)
            plsc.subcore_barrier()

            for q in range(nrows // GBB):
                pltpu.sync_copy(sh_acc.at[pl.ds(r0 + q * GBB, GBB)], rb0)
                pltpu.sync_copy(rb0, obl.at[pl.ds(r0 + q * GBB, GBB)])

        @pl.when(cid == 0)
        def _():
            chunk_pass(t0_hbm, o0_hbm, 0)
            chunk_pass(t1_hbm, o1_hbm, 1)

        @pl.when(cid == 1)
        def _():
            chunk_pass(t2_hbm, o2_hbm, 2)
            chunk_pass(t3_hbm, o3_hbm, 3)

    return k(src, dst2, alpha, t0, t1, t2, t3)


def _k15_body(dp_ref, out_ref):
    out_ref[...] = jnp.sum(dp_ref[...], axis=0) + 1e-16


def _k15(denp):
    return pl.pallas_call(
        _k15_body,
        grid=(DMR // 64,),
        in_specs=[pl.BlockSpec((NC * NS, 64, DMC), lambda i: (0, i, 0))],
        out_specs=pl.BlockSpec((64, DMC), lambda i: (i, 0)),
        out_shape=jax.ShapeDtypeStruct((DMR, DMC), jnp.float32),
    )(denp.reshape(NC * NS, DMR, DMC))


def _k2a_body(c0, c1, c2, c3, bias, stats):
    i = pl.program_id(0)
    y = jnp.concatenate([c0[...], c1[...], c2[...], c3[...]], axis=1) + bias[...]
    blk = jnp.concatenate([jnp.sum(y, axis=0, keepdims=True),
                           jnp.sum(y * y, axis=0, keepdims=True)], axis=0)

    @pl.when(i == 0)
    def _():
        stats[...] = blk

    @pl.when(i > 0)
    def _():
        stats[...] = stats[...] + blk


def _k2b_body(c0, c1, c2, c3, bias, stats1, g1, b1, wlin, w2, b2_, w3, b3_,
              z_ref, stats2):
    i = pl.program_id(0)
    y = jnp.concatenate([c0[...], c1[...], c2[...], c3[...]], axis=1) + bias[...]
    mean = stats1[0:1, :] * (1.0 / N)
    var = stats1[1:2, :] * (1.0 / N) - mean * mean
    yn = (y - mean) * lax.rsqrt(var + EPS) * g1[...] + b1[...]
    x1 = jnp.dot(yn, wlin[...], preferred_element_type=jnp.float32)
    h = jnp.maximum(jnp.dot(x1, w2[...], preferred_element_type=jnp.float32)
                    + b2_[...], 0.0)
    hh = jnp.dot(h, w3[...], preferred_element_type=jnp.float32) + b3_[...]
    z = x1 + hh
    z_ref[...] = z
    blk = jnp.concatenate([jnp.sum(z, axis=0, keepdims=True),
                           jnp.sum(z * z, axis=0, keepdims=True)], axis=0)

    @pl.when(i == 0)
    def _():
        stats2[...] = blk

    @pl.when(i > 0)
    def _():
        stats2[...] = stats2[...] + blk


def _k2c_body(z, stats2, g2, b2_, out):
    mean = stats2[0:1, :] * (1.0 / N)
    var = stats2[1:2, :] * (1.0 / N) - mean * mean
    out[...] = (z[...] - mean) * lax.rsqrt(var + EPS) * g2[...] + b2_[...]


def kernel(node_attr, edge_index, Wl, bl, Wr, br, att, bias_gat, gamma1,
           beta1, W_lin, W2, b2, W3, b3, gamma2, beta2):
    x = jnp.zeros((NP, IN_CH), jnp.float32).at[:N].set(node_attr)
    loop = jnp.arange(N, dtype=jnp.int32)
    pad = jnp.full((EP - E - N,), N, jnp.int32)
    src = jnp.concatenate([edge_index[0], loop, pad])
    dst = jnp.concatenate([edge_index[1], loop, pad])
    att_de = att.reshape(HC // 32, 16, 2).transpose(0, 2, 1).reshape(HC)

    xl, xr, t0, t1, t2, t3 = _k1(x, Wl, bl.reshape(1, HC),
                                 Wr, br.reshape(1, HC))
    xlb = lax.bitcast_convert_type(
        xl.astype(jnp.bfloat16).reshape(NP, HC // 2, 2), jnp.int32)
    xrb = lax.bitcast_convert_type(
        xr.astype(jnp.bfloat16).reshape(NP, HC // 2, 2), jnp.int32)
    ex, denp = _pass_a(xlb, xrb, src, dst, att_de)
    denm = _k15(denp).reshape(DEN_W)
    alpha = _pass_a5(dst, ex, denm)
    o0, o1, o2, o3 = _pass_b(src.reshape(EP // GBB, GBB),
                             dst.reshape(EP // GBB, GBB), alpha,
                             t0, t1, t2, t3)

    rows = 1000
    stats1 = pl.pallas_call(
        _k2a_body,
        grid=(10,),
        in_specs=[pl.BlockSpec((rows, EMB), lambda i: (i, 0))] * 4
        + [pl.BlockSpec((1, HC), lambda i: (0, 0))],
        out_specs=pl.BlockSpec((2, HC), lambda i: (0, 0)),
        out_shape=jax.ShapeDtypeStruct((2, HC), jnp.float32),
    )(o0, o1, o2, o3, bias_gat.reshape(1, HC))

    z, stats2 = pl.pallas_call(
        _k2b_body,
        grid=(10,),
        in_specs=[pl.BlockSpec((rows, EMB), lambda i: (i, 0))] * 4
        + [pl.BlockSpec((1, HC), lambda i: (0, 0)),
           pl.BlockSpec((2, HC), lambda i: (0, 0)),
           pl.BlockSpec((1, HC), lambda i: (0, 0)),
           pl.BlockSpec((1, HC), lambda i: (0, 0)),
           pl.BlockSpec((HC, EMB), lambda i: (0, 0)),
           pl.BlockSpec((EMB, FF), lambda i: (0, 0)),
           pl.BlockSpec((1, FF), lambda i: (0, 0)),
           pl.BlockSpec((FF, EMB), lambda i: (0, 0)),
           pl.BlockSpec((1, EMB), lambda i: (0, 0))],
        out_specs=[pl.BlockSpec((rows, EMB), lambda i: (i, 0)),
                   pl.BlockSpec((2, EMB), lambda i: (0, 0))],
        out_shape=[jax.ShapeDtypeStruct((N, EMB), jnp.float32),
                   jax.ShapeDtypeStruct((2, EMB), jnp.float32)],
    )(o0, o1, o2, o3, bias_gat.reshape(1, HC), stats1, gamma1.reshape(1, HC),
      beta1.reshape(1, HC), W_lin, W2, b2.reshape(1, FF), W3,
      b3.reshape(1, EMB))

    out = pl.pallas_call(
        _k2c_body,
        grid=(10,),
        in_specs=[pl.BlockSpec((rows, EMB), lambda i: (i, 0)),
                  pl.BlockSpec((2, EMB), lambda i: (0, 0)),
                  pl.BlockSpec((1, EMB), lambda i: (0, 0)),
                  pl.BlockSpec((1, EMB), lambda i: (0, 0))],
        out_specs=pl.BlockSpec((rows, EMB), lambda i: (i, 0)),
        out_shape=jax.ShapeDtypeStruct((N, EMB), jnp.float32),
    )(z, stats2, gamma2.reshape(1, EMB), beta2.reshape(1, EMB))
    return out


# speedup vs baseline: 1.1046x; 1.1046x over previous
import functools

import jax
import jax.numpy as jnp
from jax import lax
from jax.experimental import pallas as pl
from jax.experimental.pallas import tpu as pltpu
from jax.experimental.pallas import tpu_sc as plsc

N = 10000
IN_CH = 256
EMB = 128
HEADS = 4
HC = HEADS * EMB
FF = 512
NEG = 0.2
EPS = 1e-5
E = 160000

NP = 10240
EP = 172032
NC, NS, L = 2, 16, 16
TILE_A = EP // (NC * NS)
TILE_B = EP // NS
GA = 128
GS = 64
GB = 128
NBA = TILE_A // GA
NBB = TILE_B // GB
DEN_W = NP * 4
DMR, DMC = DEN_W // 128, 128


def _dyn_gather16(v, idx):
    dnums = lax.GatherDimensionNumbers(
        offset_dims=(), collapsed_slice_dims=(0,), start_index_map=(0,))
    return lax.gather(v, idx[:, None], dnums, slice_sizes=(1,),
                      mode=lax.GatherScatterMode.PROMISE_IN_BOUNDS)


def _k1_body(x_ref, wl_ref, bl_ref, wr_ref, br_ref,
             xl_ref, xr_ref, c0_ref, c1_ref, c2_ref, c3_ref):
    x = x_ref[...]
    xl = jnp.dot(x, wl_ref[...], preferred_element_type=jnp.float32) + bl_ref[...]
    xr = jnp.dot(x, wr_ref[...], preferred_element_type=jnp.float32) + br_ref[...]
    xl_ref[...] = xl
    xr_ref[...] = xr
    c0_ref[...] = xl[:, 0:128]
    c1_ref[...] = xl[:, 128:256]
    c2_ref[...] = xl[:, 256:384]
    c3_ref[...] = xl[:, 384:512]


def _k1(x, Wl, bl2, Wr, br2):
    blk = NP // 8
    return pl.pallas_call(
        _k1_body,
        grid=(8,),
        in_specs=[
            pl.BlockSpec((blk, IN_CH), lambda i: (i, 0)),
            pl.BlockSpec((IN_CH, HC), lambda i: (0, 0)),
            pl.BlockSpec((1, HC), lambda i: (0, 0)),
            pl.BlockSpec((IN_CH, HC), lambda i: (0, 0)),
            pl.BlockSpec((1, HC), lambda i: (0, 0)),
        ],
        out_specs=[
            pl.BlockSpec((blk, HC), lambda i: (i, 0)),
            pl.BlockSpec((blk, HC), lambda i: (i, 0)),
            pl.BlockSpec((blk, EMB), lambda i: (i, 0)),
            pl.BlockSpec((blk, EMB), lambda i: (i, 0)),
            pl.BlockSpec((blk, EMB), lambda i: (i, 0)),
            pl.BlockSpec((blk, EMB), lambda i: (i, 0)),
        ],
        out_shape=[
            jax.ShapeDtypeStruct((NP, HC), jnp.float32),
            jax.ShapeDtypeStruct((NP, HC), jnp.float32),
            jax.ShapeDtypeStruct((NP, EMB), jnp.float32),
            jax.ShapeDtypeStruct((NP, EMB), jnp.float32),
            jax.ShapeDtypeStruct((NP, EMB), jnp.float32),
            jax.ShapeDtypeStruct((NP, EMB), jnp.float32),
        ],
    )(x, Wl, bl2, Wr, br2)


def _pass_a(xl, xr, src, dst, att_flat):
    mesh = plsc.VectorSubcoreMesh(core_axis_name="c", subcore_axis_name="s")
    HW = HC // 2

    @functools.partial(
        pl.kernel,
        mesh=mesh,
        compiler_params=pltpu.CompilerParams(needs_layout_passes=False),
        out_type=[
            jax.ShapeDtypeStruct((4, EP), jnp.float32),
            jax.ShapeDtypeStruct((NC * NS * DEN_W,), jnp.float32),
        ],
        scratch_types=[
            pltpu.VMEM((128,), jnp.int32),
            pltpu.VMEM((128,), jnp.int32),
            pltpu.VMEM((128,), jnp.int32),
            pltpu.VMEM((128,), jnp.int32),
            pltpu.VMEM((GS, HW), jnp.int32),
            pltpu.VMEM((GS, HW), jnp.int32),
            pltpu.VMEM((GS, HW), jnp.int32),
            pltpu.VMEM((GS, HW), jnp.int32),
            pltpu.VMEM((4, 256), jnp.float32),
            pltpu.VMEM((HC,), jnp.float32),
            pltpu.VMEM((DEN_W,), jnp.float32),
            pltpu.SemaphoreType.DMA,
            pltpu.SemaphoreType.DMA,
            pltpu.SemaphoreType.DMA,
            pltpu.SemaphoreType.DMA,
            pltpu.SemaphoreType.DMA,
            pltpu.SemaphoreType.DMA,
            pltpu.SemaphoreType.DMA,
            pltpu.SemaphoreType.DMA,
        ],
    )
    def k(xl_hbm, xr_hbm, src_hbm, dst_hbm, att_hbm, ex_hbm, den_hbm,
          ids0, ids1, idd0, idd1, rl0, rl1, rr0, rr1, ex_buf, att_v, den_v,
          sis0, sis1, sid0, sid1, sl0, sl1, sr0, sr1):
        cid = lax.axis_index("c")
        sid = lax.axis_index("s")
        tid = sid * NC + cid
        ii = lax.iota(jnp.int32, L)
        zv = jnp.zeros((L,), jnp.float32)
        IDS, IDD = (ids0, ids1), (idd0, idd1)
        SIS, SID = (sis0, sis1), (sid0, sid1)
        RL, RR = (rl0, rl1), (rr0, rr1)
        SL, SR = (sl0, sl1), (sr0, sr1)

        def zb(i, _):
            den_v[pl.ds(lax.mul(i, L), L)] = zv
            return 0
        lax.fori_loop(0, DEN_W // L, zb, 0)

        pltpu.sync_copy(att_hbm, att_v)
        att_e = [[att_v[pl.ds(h * EMB + w * 2 * L, L)] for w in range(4)]
                 for h in range(HEADS)]
        att_o = [[att_v[pl.ds(h * EMB + w * 2 * L + L, L)] for w in range(4)]
                 for h in range(HEADS)]

        ebase = lax.mul(tid, TILE_A)

        def issue_idx(jq, p):
            off = jnp.minimum(ebase + jq * 128, EP - 128)
            pltpu.async_copy(src_hbm.at[pl.ds(off, 128)], IDS[p], SIS[p])
            pltpu.async_copy(dst_hbm.at[pl.ds(off, 128)], IDD[p], SID[p])

        def wait_idx(p):
            pltpu.make_async_copy(
                src_hbm.at[pl.ds(0, 128)], IDS[p], SIS[p]).wait()
            pltpu.make_async_copy(
                dst_hbm.at[pl.ds(0, 128)], IDD[p], SID[p]).wait()

        def issue_g(half, b, pidx):
            pltpu.async_copy(
                xl_hbm.at[IDS[pidx].at[pl.ds(half * GS, GS)]], RL[b], SL[b])
            pltpu.async_copy(
                xr_hbm.at[IDD[pidx].at[pl.ds(half * GS, GS)]], RR[b], SR[b])

        def wait_g(b):
            pltpu.make_async_copy(
                xl_hbm.at[pl.ds(0, GS)], RL[b], SL[b]).wait()
            pltpu.make_async_copy(
                xr_hbm.at[pl.ds(0, GS)], RR[b], SR[b]).wait()

        def compute(half, b, pidx, exoff):
            rl, rr = RL[b], RR[b]
            idd = IDD[pidx]

            def subgrp(sg, _):
                colq = half * GS + sg * L
                col = exoff + colq

                def edge(g, lv):
                    gg = sg * L + g
                    sel = ii == g
                    for h in range(HEADS):
                        acc = jnp.zeros((L,), jnp.float32)
                        for w in range(4):
                            off = h * (EMB // 2) + w * L
                            wl = rl[gg, pl.ds(off, L)]
                            wr = rr[gg, pl.ds(off, L)]
                            le = plsc.bitcast(
                                lax.shift_left(wl, 16), jnp.float32)
                            lo = plsc.bitcast(
                                lax.bitwise_and(wl, -65536), jnp.float32)
                            re_ = plsc.bitcast(
                                lax.shift_left(wr, 16), jnp.float32)
                            ro = plsc.bitcast(
                                lax.bitwise_and(wr, -65536), jnp.float32)
                            te = le + re_
                            te = jnp.maximum(te, NEG * te)
                            acc = acc + te * att_e[h][w]
                            to = lo + ro
                            to = jnp.maximum(to, NEG * to)
                            acc = acc + to * att_o[h][w]
                        red = acc
                        for st in (8, 4, 2, 1):
                            red = red + _dyn_gather16(
                                red, lax.bitwise_xor(ii, st))
                        lv = (lv[:h] + (jnp.where(sel, red, lv[h]),)
                              + lv[h + 1:])
                    return lv
                lv = lax.fori_loop(0, L, edge, (zv, zv, zv, zv))
                dv = idd[pl.ds(colq, L)]
                for h in range(HEADS):
                    ev = jnp.exp(lv[h])
                    ex_buf[h, pl.ds(col, L)] = ev
                    plsc.addupdate_scatter(den_v, [dv * 4 + h], ev)
                return 0
            lax.fori_loop(0, GS // L, subgrp, 0)

        issue_idx(0, 0)
        issue_idx(1, 1)
        wait_idx(0)
        issue_g(0, 0, 0)
        issue_g(1, 1, 0)

        def pair(m, _):
            a2 = m * 2
            wait_g(0)
            compute(0, 0, 0, 0)
            wait_idx(1)
            issue_g(0, 0, 1)
            wait_g(1)
            compute(1, 1, 0, 0)
            issue_g(1, 1, 1)
            issue_idx(a2 + 2, 0)
            wait_g(0)
            compute(0, 0, 1, 128)
            wait_idx(0)
            issue_g(0, 0, 0)
            wait_g(1)
            compute(1, 1, 1, 128)
            issue_g(1, 1, 0)
            issue_idx(a2 + 3, 1)
            eb = ebase + m * 256
            for h in range(HEADS):
                pltpu.sync_copy(ex_buf.at[h],
                                ex_hbm.at[h].at[pl.ds(eb, 256)])
            return 0
        lax.fori_loop(0, TILE_A // 256, pair, 0)
        wait_g(0)
        wait_g(1)
        wait_idx(1)

        pltpu.sync_copy(den_v,
                        den_hbm.at[pl.ds(lax.mul(tid, DEN_W), DEN_W)])

    return k(xl, xr, src, dst, att_flat)


A5B = 384


def _pass_a5(dst, ex, denm):
    mesh = plsc.VectorSubcoreMesh(core_axis_name="c", subcore_axis_name="s")

    @functools.partial(
        pl.kernel,
        mesh=mesh,
        compiler_params=pltpu.CompilerParams(needs_layout_passes=False),
        out_type=jax.ShapeDtypeStruct((4, EP), jnp.float32),
        scratch_types=[
            pltpu.VMEM((A5B,), jnp.int32),
            pltpu.VMEM((4, A5B), jnp.float32),
            pltpu.VMEM((4, A5B), jnp.float32),
            pltpu.VMEM((DEN_W,), jnp.float32),
        ],
    )
    def k(dst_hbm, ex_hbm, den_hbm, al_hbm, idx_d, exb, alb, d_v):
        cid = lax.axis_index("c")
        sid = lax.axis_index("s")
        tid = sid * NC + cid
        ii = lax.iota(jnp.int32, L)
        pltpu.sync_copy(den_hbm, d_v)
        ebase = lax.mul(tid, TILE_A)

        def batch(b, _):
            eb = ebase + b * A5B
            pltpu.sync_copy(dst_hbm.at[pl.ds(eb, A5B)], idx_d)
            for h in range(HEADS):
                pltpu.sync_copy(ex_hbm.at[h].at[pl.ds(eb, A5B)],
                                exb.at[h].at[pl.ds(0, A5B)])
            for sub in range(0, A5B, L):
                dv = idx_d[pl.ds(sub, L)]
                for h in range(HEADS):
                    fi = dv * 4 + h
                    den = plsc.load_gather(d_v, [fi])
                    alb[h, pl.ds(sub, L)] = exb[h, pl.ds(sub, L)] / den
            for h in range(HEADS):
                pltpu.sync_copy(alb.at[h].at[pl.ds(0, A5B)],
                                al_hbm.at[h].at[pl.ds(eb, A5B)])
            return 0
        lax.fori_loop(0, TILE_A // A5B, batch, 0)

    return k(dst, ex, denm)


GBB = 64
BLK_B = 512


def _pass_b(src, dst2, alpha, t0, t1, t2, t3):
    mesh = plsc.VectorSubcoreMesh(core_axis_name="c", subcore_axis_name="s")

    @functools.partial(
        pl.kernel,
        mesh=mesh,
        compiler_params=pltpu.CompilerParams(needs_layout_passes=False),
        out_type=[jax.ShapeDtypeStruct((NP, EMB), jnp.float32)
                  for _ in range(4)],
        scratch_types=[
            pltpu.VMEM((BLK_B // GBB, GBB), jnp.int32),
            pltpu.VMEM((BLK_B // GBB, GBB), jnp.int32),
            pltpu.VMEM((BLK_B,), jnp.float32),
            pltpu.VMEM((GBB, EMB), jnp.float32),
            pltpu.VMEM((GBB, EMB), jnp.float32),
            pltpu.VMEM((GBB, EMB), jnp.float32),
            pltpu.VMEM((GBB, EMB), jnp.float32),
            pltpu.VMEM_SHARED((NP, EMB), jnp.float32),
            pltpu.SemaphoreType.DMA,
            pltpu.SemaphoreType.DMA,
            pltpu.SemaphoreType.DMA,
            pltpu.SemaphoreType.DMA,
            pltpu.SemaphoreType.DMA,
            pltpu.SemaphoreType.DMA,
            pltpu.SemaphoreType.DMA,
            pltpu.SemaphoreType.DMA,
        ],
    )
    def k(src2_hbm, dst2_hbm, al_hbm, t0_hbm, t1_hbm, t2_hbm, t3_hbm,
          o0_hbm, o1_hbm, o2_hbm, o3_hbm,
          idx_s2, idx_d2, alb, rb0, rb1, rb2, rb3, sh_acc,
          sg0, sg1, sg2, sg3, ss0, ss1, ss2, ss3):
        cid = lax.axis_index("c")
        sid = lax.axis_index("s")
        ii = lax.iota(jnp.int32, L)
        zv = jnp.zeros((L,), jnp.float32)
        RB = (rb0, rb1, rb2, rb3)
        SG = (sg0, sg1, sg2, sg3)
        SS = (ss0, ss1, ss2, ss3)

        nrows = NP // NS
        r0 = lax.mul(sid, nrows)
        ebase = lax.mul(sid, TILE_B)
        rbase = lax.mul(sid, TILE_B // GBB)

        def chunk_pass(tbl, obl, c):
            def zr(i, _):
                rb0[lax.shift_right_logical(i, 3),
                    pl.ds(lax.mul(lax.bitwise_and(i, 7), L), L)] = zv
                return 0
            lax.fori_loop(0, GBB * (EMB // L), zr, 0)
            for q in range(nrows // GBB):
                pltpu.sync_copy(rb0, sh_acc.at[pl.ds(r0 + q * GBB, GBB)])
            plsc.subcore_barrier()

            def issue_g(ib, p):
                return pltpu.async_copy(
                    tbl.at[idx_s2.at[ib]], RB[p], SG[p])

            def block(b, _):
                eb = ebase + b * BLK_B
                rr = rbase + b * (BLK_B // GBB)
                pltpu.sync_copy(src2_hbm.at[pl.ds(rr, BLK_B // GBB)], idx_s2)
                pltpu.sync_copy(dst2_hbm.at[pl.ds(rr, BLK_B // GBB)], idx_d2)
                pltpu.sync_copy(al_hbm.at[c].at[pl.ds(eb, BLK_B)], alb)
                hg = [issue_g(0, 0), issue_g(1, 1), None, None]
                hs = [None, None, None, None]
                for ib in range(BLK_B // GBB):
                    p = ib & 3
                    hg[p].wait()
                    rows = RB[p]
                    for sub in range(0, GBB, L):
                        av = alb[pl.ds(ib * GBB + sub, L)]

                        def edge(g, _):
                            gg = sub + g
                            bc = _dyn_gather16(av, jnp.full((L,), g, jnp.int32))
                            for j in range(EMB // L):
                                rows[gg, pl.ds(j * L, L)] = (
                                    rows[gg, pl.ds(j * L, L)] * bc)
                            return 0
                        lax.fori_loop(0, L, edge, 0)
                    hs[p] = pltpu.async_copy(
                        rows, sh_acc.at[idx_d2.at[ib]], SS[p], add=True)
                    if ib < BLK_B // GBB - 2:
                        pn = (ib + 2) & 3
                        if hs[pn] is not None:
                            hs[pn].wait()
                        hg[pn] = issue_g(ib + 2, pn)
                for p in range(4):
                    hs[p].wait()
                return 0
            lax.fori_loop(0, TILE_B // BLK_B, block, 0)
            plsc.subcore_barrier()

            for q in range(nrows // GBB):
                pltpu.sync_copy(sh_acc.at[pl.ds(r0 + q * GBB, GBB)], rb0)
                pltpu.sync_copy(rb0, obl.at[pl.ds(r0 + q * GBB, GBB)])

        @pl.when(cid == 0)
        def _():
            chunk_pass(t0_hbm, o0_hbm, 0)
            chunk_pass(t1_hbm, o1_hbm, 1)

        @pl.when(cid == 1)
        def _():
            chunk_pass(t2_hbm, o2_hbm, 2)
            chunk_pass(t3_hbm, o3_hbm, 3)

    return k(src, dst2, alpha, t0, t1, t2, t3)


def _k15_body(dp_ref, out_ref):
    out_ref[...] = jnp.sum(dp_ref[...], axis=0) + 1e-16


def _k15(denp):
    return pl.pallas_call(
        _k15_body,
        grid=(DMR // 64,),
        in_specs=[pl.BlockSpec((NC * NS, 64, DMC), lambda i: (0, i, 0))],
        out_specs=pl.BlockSpec((64, DMC), lambda i: (i, 0)),
        out_shape=jax.ShapeDtypeStruct((DMR, DMC), jnp.float32),
    )(denp.reshape(NC * NS, DMR, DMC))


def _k2a_body(c0, c1, c2, c3, bias, stats):
    i = pl.program_id(0)
    y = jnp.concatenate([c0[...], c1[...], c2[...], c3[...]], axis=1) + bias[...]
    blk = jnp.concatenate([jnp.sum(y, axis=0, keepdims=True),
                           jnp.sum(y * y, axis=0, keepdims=True)], axis=0)

    @pl.when(i == 0)
    def _():
        stats[...] = blk

    @pl.when(i > 0)
    def _():
        stats[...] = stats[...] + blk


def _k2b_body(c0, c1, c2, c3, bias, stats1, g1, b1, wlin, w2, b2_, w3, b3_,
              z_ref, stats2):
    i = pl.program_id(0)
    y = jnp.concatenate([c0[...], c1[...], c2[...], c3[...]], axis=1) + bias[...]
    mean = stats1[0:1, :] * (1.0 / N)
    var = stats1[1:2, :] * (1.0 / N) - mean * mean
    yn = (y - mean) * lax.rsqrt(var + EPS) * g1[...] + b1[...]
    x1 = jnp.dot(yn, wlin[...], preferred_element_type=jnp.float32)
    h = jnp.maximum(jnp.dot(x1, w2[...], preferred_element_type=jnp.float32)
                    + b2_[...], 0.0)
    hh = jnp.dot(h, w3[...], preferred_element_type=jnp.float32) + b3_[...]
    z = x1 + hh
    z_ref[...] = z
    blk = jnp.concatenate([jnp.sum(z, axis=0, keepdims=True),
                           jnp.sum(z * z, axis=0, keepdims=True)], axis=0)

    @pl.when(i == 0)
    def _():
        stats2[...] = blk

    @pl.when(i > 0)
    def _():
        stats2[...] = stats2[...] + blk


def _k2c_body(z, stats2, g2, b2_, out):
    mean = stats2[0:1, :] * (1.0 / N)
    var = stats2[1:2, :] * (1.0 / N) - mean * mean
    out[...] = (z[...] - mean) * lax.rsqrt(var + EPS) * g2[...] + b2_[...]


def kernel(node_attr, edge_index, Wl, bl, Wr, br, att, bias_gat, gamma1,
           beta1, W_lin, W2, b2, W3, b3, gamma2, beta2):
    x = jnp.zeros((NP, IN_CH), jnp.float32).at[:N].set(node_attr)
    loop = jnp.arange(N, dtype=jnp.int32)
    pad = jnp.full((EP - E - N,), N, jnp.int32)
    src = jnp.concatenate([edge_index[0], loop, pad])
    dst = jnp.concatenate([edge_index[1], loop, pad])
    att_de = att.reshape(HC // 32, 16, 2).transpose(0, 2, 1).reshape(HC)

    xl, xr, t0, t1, t2, t3 = _k1(x, Wl, bl.reshape(1, HC),
                                 Wr, br.reshape(1, HC))
    xlb = lax.bitcast_convert_type(
        xl.astype(jnp.bfloat16).reshape(NP, HC // 2, 2), jnp.int32)
    xrb = lax.bitcast_convert_type(
        xr.astype(jnp.bfloat16).reshape(NP, HC // 2, 2), jnp.int32)
    ex, denp = _pass_a(xlb, xrb, src, dst, att_de)
    denm = _k15(denp).reshape(DEN_W)
    alpha = _pass_a5(dst, ex, denm)
    o0, o1, o2, o3 = _pass_b(src.reshape(EP // GBB, GBB),
                             dst.reshape(EP // GBB, GBB), alpha,
                             t0, t1, t2, t3)

    rows = 1000
    stats1 = pl.pallas_call(
        _k2a_body,
        grid=(10,),
        in_specs=[pl.BlockSpec((rows, EMB), lambda i: (i, 0))] * 4
        + [pl.BlockSpec((1, HC), lambda i: (0, 0))],
        out_specs=pl.BlockSpec((2, HC), lambda i: (0, 0)),
        out_shape=jax.ShapeDtypeStruct((2, HC), jnp.float32),
    )(o0, o1, o2, o3, bias_gat.reshape(1, HC))

    z, stats2 = pl.pallas_call(
        _k2b_body,
        grid=(10,),
        in_specs=[pl.BlockSpec((rows, EMB), lambda i: (i, 0))] * 4
        + [pl.BlockSpec((1, HC), lambda i: (0, 0)),
           pl.BlockSpec((2, HC), lambda i: (0, 0)),
           pl.BlockSpec((1, HC), lambda i: (0, 0)),
           pl.BlockSpec((1, HC), lambda i: (0, 0)),
           pl.BlockSpec((HC, EMB), lambda i: (0, 0)),
           pl.BlockSpec((EMB, FF), lambda i: (0, 0)),
           pl.BlockSpec((1, FF), lambda i: (0, 0)),
           pl.BlockSpec((FF, EMB), lambda i: (0, 0)),
           pl.BlockSpec((1, EMB), lambda i: (0, 0))],
        out_specs=[pl.BlockSpec((rows, EMB), lambda i: (i, 0)),
                   pl.BlockSpec((2, EMB), lambda i: (0, 0))],
        out_shape=[jax.ShapeDtypeStruct((N, EMB), jnp.float32),
                   jax.ShapeDtypeStruct((2, EMB), jnp.float32)],
    )(o0, o1, o2, o3, bias_gat.reshape(1, HC), stats1, gamma1.reshape(1, HC),
      beta1.reshape(1, HC), W_lin, W2, b2.reshape(1, FF), W3,
      b3.reshape(1, EMB))

    out = pl.pallas_call(
        _k2c_body,
        grid=(10,),
        in_specs=[pl.BlockSpec((rows, EMB), lambda i: (i, 0)),
                  pl.BlockSpec((2, EMB), lambda i: (0, 0)),
                  pl.BlockSpec((1, EMB), lambda i: (0, 0)),
                  pl.BlockSpec((1, EMB), lambda i: (0, 0))],
        out_specs=pl.BlockSpec((rows, EMB), lambda i: (i, 0)),
        out_shape=jax.ShapeDtypeStruct((N, EMB), jnp.float32),
    )(z, stats2, gamma2.reshape(1, EMB), beta2.reshape(1, EMB))
    return out


# speedup vs baseline: 1.1589x; 1.0491x over previous
import functools

import jax
import jax.numpy as jnp
from jax import lax
from jax.experimental import pallas as pl
from jax.experimental.pallas import tpu as pltpu
from jax.experimental.pallas import tpu_sc as plsc

N = 10000
IN_CH = 256
EMB = 128
HEADS = 4
HC = HEADS * EMB
FF = 512
NEG = 0.2
EPS = 1e-5
E = 160000

NP = 10240
EP = 172032
NC, NS, L = 2, 16, 16
TILE_A = EP // (NC * NS)
TILE_B = EP // NS
GA = 128
GS = 64
GB = 128
NBA = TILE_A // GA
NBB = TILE_B // GB
DEN_W = NP * 4
DMR, DMC = DEN_W // 128, 128


def _dyn_gather16(v, idx):
    dnums = lax.GatherDimensionNumbers(
        offset_dims=(), collapsed_slice_dims=(0,), start_index_map=(0,))
    return lax.gather(v, idx[:, None], dnums, slice_sizes=(1,),
                      mode=lax.GatherScatterMode.PROMISE_IN_BOUNDS)


def _k1_body(x_ref, wl_ref, bl_ref, wr_ref, br_ref,
             xl_ref, xr_ref, c0_ref, c1_ref, c2_ref, c3_ref):
    x = x_ref[...]
    xl = jnp.dot(x, wl_ref[...], preferred_element_type=jnp.float32) + bl_ref[...]
    xr = jnp.dot(x, wr_ref[...], preferred_element_type=jnp.float32) + br_ref[...]
    xl_ref[...] = xl
    xr_ref[...] = xr
    c0_ref[...] = xl[:, 0:128]
    c1_ref[...] = xl[:, 128:256]
    c2_ref[...] = xl[:, 256:384]
    c3_ref[...] = xl[:, 384:512]


def _k1(x, Wl, bl2, Wr, br2):
    blk = NP // 8
    return pl.pallas_call(
        _k1_body,
        grid=(8,),
        in_specs=[
            pl.BlockSpec((blk, IN_CH), lambda i: (i, 0)),
            pl.BlockSpec((IN_CH, HC), lambda i: (0, 0)),
            pl.BlockSpec((1, HC), lambda i: (0, 0)),
            pl.BlockSpec((IN_CH, HC), lambda i: (0, 0)),
            pl.BlockSpec((1, HC), lambda i: (0, 0)),
        ],
        out_specs=[
            pl.BlockSpec((blk, HC), lambda i: (i, 0)),
            pl.BlockSpec((blk, HC), lambda i: (i, 0)),
            pl.BlockSpec((blk, EMB), lambda i: (i, 0)),
            pl.BlockSpec((blk, EMB), lambda i: (i, 0)),
            pl.BlockSpec((blk, EMB), lambda i: (i, 0)),
            pl.BlockSpec((blk, EMB), lambda i: (i, 0)),
        ],
        out_shape=[
            jax.ShapeDtypeStruct((NP, HC), jnp.float32),
            jax.ShapeDtypeStruct((NP, HC), jnp.float32),
            jax.ShapeDtypeStruct((NP, EMB), jnp.float32),
            jax.ShapeDtypeStruct((NP, EMB), jnp.float32),
            jax.ShapeDtypeStruct((NP, EMB), jnp.float32),
            jax.ShapeDtypeStruct((NP, EMB), jnp.float32),
        ],
    )(x, Wl, bl2, Wr, br2)


def _pass_a(xl, xr, src, dst, att_flat):
    mesh = plsc.VectorSubcoreMesh(core_axis_name="c", subcore_axis_name="s")
    HW = HC // 2

    @functools.partial(
        pl.kernel,
        mesh=mesh,
        compiler_params=pltpu.CompilerParams(needs_layout_passes=False),
        out_type=[
            jax.ShapeDtypeStruct((4, EP), jnp.float32),
            jax.ShapeDtypeStruct((NC * NS * DEN_W,), jnp.float32),
        ],
        scratch_types=[
            pltpu.VMEM((128,), jnp.int32),
            pltpu.VMEM((128,), jnp.int32),
            pltpu.VMEM((128,), jnp.int32),
            pltpu.VMEM((128,), jnp.int32),
            pltpu.VMEM((GS, HW), jnp.int32),
            pltpu.VMEM((GS, HW), jnp.int32),
            pltpu.VMEM((GS, HW), jnp.int32),
            pltpu.VMEM((GS, HW), jnp.int32),
            pltpu.VMEM((4, 256), jnp.float32),
            pltpu.VMEM((HC,), jnp.float32),
            pltpu.VMEM((DEN_W,), jnp.float32),
            pltpu.SemaphoreType.DMA,
            pltpu.SemaphoreType.DMA,
            pltpu.SemaphoreType.DMA,
            pltpu.SemaphoreType.DMA,
            pltpu.SemaphoreType.DMA,
            pltpu.SemaphoreType.DMA,
            pltpu.SemaphoreType.DMA,
            pltpu.SemaphoreType.DMA,
        ],
    )
    def k(xl_hbm, xr_hbm, src_hbm, dst_hbm, att_hbm, ex_hbm, den_hbm,
          ids0, ids1, idd0, idd1, rl0, rl1, rr0, rr1, ex_buf, att_v, den_v,
          sis0, sis1, sid0, sid1, sl0, sl1, sr0, sr1):
        cid = lax.axis_index("c")
        sid = lax.axis_index("s")
        tid = sid * NC + cid
        ii = lax.iota(jnp.int32, L)
        zv = jnp.zeros((L,), jnp.float32)
        IDS, IDD = (ids0, ids1), (idd0, idd1)
        SIS, SID = (sis0, sis1), (sid0, sid1)
        RL, RR = (rl0, rl1), (rr0, rr1)
        SL, SR = (sl0, sl1), (sr0, sr1)

        def zb(i, _):
            den_v[pl.ds(lax.mul(i, L), L)] = zv
            return 0
        lax.fori_loop(0, DEN_W // L, zb, 0)

        pltpu.sync_copy(att_hbm, att_v)
        att_e = [[att_v[pl.ds(h * EMB + w * 2 * L, L)] for w in range(4)]
                 for h in range(HEADS)]
        att_o = [[att_v[pl.ds(h * EMB + w * 2 * L + L, L)] for w in range(4)]
                 for h in range(HEADS)]

        ebase = lax.mul(tid, TILE_A)

        def issue_idx(jq, p):
            off = jnp.minimum(ebase + jq * 128, EP - 128)
            pltpu.async_copy(src_hbm.at[pl.ds(off, 128)], IDS[p], SIS[p])
            pltpu.async_copy(dst_hbm.at[pl.ds(off, 128)], IDD[p], SID[p])

        def wait_idx(p):
            pltpu.make_async_copy(
                src_hbm.at[pl.ds(0, 128)], IDS[p], SIS[p]).wait()
            pltpu.make_async_copy(
                dst_hbm.at[pl.ds(0, 128)], IDD[p], SID[p]).wait()

        def issue_g(half, b, pidx):
            pltpu.async_copy(
                xl_hbm.at[IDS[pidx].at[pl.ds(half * GS, GS)]], RL[b], SL[b])
            pltpu.async_copy(
                xr_hbm.at[IDD[pidx].at[pl.ds(half * GS, GS)]], RR[b], SR[b])

        def wait_g(b):
            pltpu.make_async_copy(
                xl_hbm.at[pl.ds(0, GS)], RL[b], SL[b]).wait()
            pltpu.make_async_copy(
                xr_hbm.at[pl.ds(0, GS)], RR[b], SR[b]).wait()

        def compute(half, b, pidx, exoff):
            rl, rr = RL[b], RR[b]
            idd = IDD[pidx]

            def subgrp(sg, _):
                colq = half * GS + sg * L
                col = exoff + colq

                def edge(g, lv):
                    gg = sg * L + g
                    sel = ii == g
                    for h in range(HEADS):
                        acc = jnp.zeros((L,), jnp.float32)
                        for w in range(4):
                            off = h * (EMB // 2) + w * L
                            wl = rl[gg, pl.ds(off, L)]
                            wr = rr[gg, pl.ds(off, L)]
                            le = plsc.bitcast(
                                lax.shift_left(wl, 16), jnp.float32)
                            lo = plsc.bitcast(
                                lax.bitwise_and(wl, -65536), jnp.float32)
                            re_ = plsc.bitcast(
                                lax.shift_left(wr, 16), jnp.float32)
                            ro = plsc.bitcast(
                                lax.bitwise_and(wr, -65536), jnp.float32)
                            te = le + re_
                            te = jnp.maximum(te, NEG * te)
                            acc = acc + te * att_e[h][w]
                            to = lo + ro
                            to = jnp.maximum(to, NEG * to)
                            acc = acc + to * att_o[h][w]
                        red = acc
                        for st in (8, 4, 2, 1):
                            red = red + _dyn_gather16(
                                red, lax.bitwise_xor(ii, st))
                        lv = (lv[:h] + (jnp.where(sel, red, lv[h]),)
                              + lv[h + 1:])
                    return lv
                lv = lax.fori_loop(0, L, edge, (zv, zv, zv, zv))
                dv = idd[pl.ds(colq, L)]
                for h in range(HEADS):
                    ev = jnp.exp(lv[h])
                    ex_buf[h, pl.ds(col, L)] = ev
                    plsc.addupdate_scatter(den_v, [dv * 4 + h], ev)
                return 0
            lax.fori_loop(0, GS // L, subgrp, 0)

        issue_idx(0, 0)
        issue_idx(1, 1)
        wait_idx(0)
        issue_g(0, 0, 0)
        issue_g(1, 1, 0)

        def pair(m, _):
            a2 = m * 2
            wait_g(0)
            compute(0, 0, 0, 0)
            wait_idx(1)
            issue_g(0, 0, 1)
            wait_g(1)
            compute(1, 1, 0, 0)
            issue_g(1, 1, 1)
            issue_idx(a2 + 2, 0)
            wait_g(0)
            compute(0, 0, 1, 128)
            wait_idx(0)
            issue_g(0, 0, 0)
            wait_g(1)
            compute(1, 1, 1, 128)
            issue_g(1, 1, 0)
            issue_idx(a2 + 3, 1)
            eb = ebase + m * 256
            for h in range(HEADS):
                pltpu.sync_copy(ex_buf.at[h],
                                ex_hbm.at[h].at[pl.ds(eb, 256)])
            return 0
        lax.fori_loop(0, TILE_A // 256, pair, 0)
        wait_g(0)
        wait_g(1)
        wait_idx(1)

        pltpu.sync_copy(den_v,
                        den_hbm.at[pl.ds(lax.mul(tid, DEN_W), DEN_W)])

    return k(xl, xr, src, dst, att_flat)


A5B = 384


def _pass_a5(dst, ex, denm):
    mesh = plsc.VectorSubcoreMesh(core_axis_name="c", subcore_axis_name="s")

    @functools.partial(
        pl.kernel,
        mesh=mesh,
        compiler_params=pltpu.CompilerParams(needs_layout_passes=False),
        out_type=jax.ShapeDtypeStruct((4, EP), jnp.float32),
        scratch_types=[
            pltpu.VMEM((A5B,), jnp.int32),
            pltpu.VMEM((4, A5B), jnp.float32),
            pltpu.VMEM((4, A5B), jnp.float32),
            pltpu.VMEM((DEN_W,), jnp.float32),
        ],
    )
    def k(dst_hbm, ex_hbm, den_hbm, al_hbm, idx_d, exb, alb, d_v):
        cid = lax.axis_index("c")
        sid = lax.axis_index("s")
        tid = sid * NC + cid
        ii = lax.iota(jnp.int32, L)
        pltpu.sync_copy(den_hbm, d_v)
        ebase = lax.mul(tid, TILE_A)

        def batch(b, _):
            eb = ebase + b * A5B
            pltpu.sync_copy(dst_hbm.at[pl.ds(eb, A5B)], idx_d)
            for h in range(HEADS):
                pltpu.sync_copy(ex_hbm.at[h].at[pl.ds(eb, A5B)],
                                exb.at[h].at[pl.ds(0, A5B)])
            for sub in range(0, A5B, L):
                dv = idx_d[pl.ds(sub, L)]
                for h in range(HEADS):
                    fi = dv * 4 + h
                    den = plsc.load_gather(d_v, [fi])
                    alb[h, pl.ds(sub, L)] = exb[h, pl.ds(sub, L)] / den
            for h in range(HEADS):
                pltpu.sync_copy(alb.at[h].at[pl.ds(0, A5B)],
                                al_hbm.at[h].at[pl.ds(eb, A5B)])
            return 0
        lax.fori_loop(0, TILE_A // A5B, batch, 0)

    return k(dst, ex, denm)


GBB = 64
BLK_B = 1536


def _pass_b(src2, dst2, alpha, t0, t1, t2, t3):
    mesh = plsc.VectorSubcoreMesh(core_axis_name="c", subcore_axis_name="s")
    CW = EMB // 2

    @functools.partial(
        pl.kernel,
        mesh=mesh,
        compiler_params=pltpu.CompilerParams(needs_layout_passes=False),
        out_type=[jax.ShapeDtypeStruct((NP, EMB), jnp.float32)
                  for _ in range(4)],
        scratch_types=[
            pltpu.VMEM((BLK_B // GBB, GBB), jnp.int32),
            pltpu.VMEM((BLK_B // GBB, GBB), jnp.int32),
            pltpu.VMEM((BLK_B,), jnp.float32),
            pltpu.VMEM((GBB, EMB), jnp.float32),
            pltpu.VMEM((GBB, EMB), jnp.float32),
            pltpu.VMEM((GBB, EMB), jnp.float32),
            pltpu.VMEM((GBB, EMB), jnp.float32),
            pltpu.VMEM_SHARED((NP, EMB), jnp.float32),
            pltpu.SemaphoreType.DMA,
            pltpu.SemaphoreType.DMA,
            pltpu.SemaphoreType.DMA,
            pltpu.SemaphoreType.DMA,
            pltpu.SemaphoreType.DMA,
            pltpu.SemaphoreType.DMA,
            pltpu.SemaphoreType.DMA,
            pltpu.SemaphoreType.DMA,
        ],
    )
    def k(src2_hbm, dst2_hbm, al_hbm, t0_hbm, t1_hbm, t2_hbm, t3_hbm,
          o0_hbm, o1_hbm, o2_hbm, o3_hbm,
          idx_s2, idx_d2, alb, sb0, sb1, sb2, sb3, sh_acc,
          sg0, sg1, sg2, sg3, ss0, ss1, ss2, ss3):
        cid = lax.axis_index("c")
        sid = lax.axis_index("s")
        ii = lax.iota(jnp.int32, L)
        zv = jnp.zeros((L,), jnp.float32)
        SB = (sb0, sb1, sb2, sb3)
        SG = (sg0, sg1, sg2, sg3)
        SS = (ss0, ss1, ss2, ss3)
        NB = BLK_B // GBB

        nrows = NP // NS
        r0 = lax.mul(sid, nrows)
        ebase = lax.mul(sid, TILE_B)
        rbase = lax.mul(sid, TILE_B // GBB)

        def chunk_pass(tbl, obl, c):
            def zr(i, _):
                sb0[lax.shift_right_logical(i, 3),
                    pl.ds(lax.mul(lax.bitwise_and(i, 7), L), L)] = zv
                return 0
            lax.fori_loop(0, GBB * (EMB // L), zr, 0)
            for q in range(nrows // GBB):
                pltpu.sync_copy(sb0, sh_acc.at[pl.ds(r0 + q * GBB, GBB)])
            plsc.subcore_barrier()

            def issue_g(ib, p):
                return pltpu.async_copy(
                    tbl.at[idx_s2.at[ib]], SB[p], SG[p])

            def block(b, _):
                eb = ebase + b * BLK_B
                rr = rbase + b * NB
                pltpu.sync_copy(src2_hbm.at[pl.ds(rr, NB)], idx_s2)
                pltpu.sync_copy(dst2_hbm.at[pl.ds(rr, NB)], idx_d2)
                pltpu.sync_copy(al_hbm.at[c].at[pl.ds(eb, BLK_B)], alb)
                hg = [issue_g(0, 0), issue_g(1, 1), None, None]
                hs = [None, None, None, None]
                for ib in range(NB):
                    ps = ib & 3
                    hg[ps].wait()
                    sbuf = SB[ps]

                    def subgrp(sg_, _):
                        base16 = lax.mul(sg_, L)
                        av = alb[pl.ds(ib * GBB + base16, L)]

                        def edge(g, _):
                            gg = base16 + g
                            bc = _dyn_gather16(
                                av, jnp.full((L,), g, jnp.int32))
                            for j in range(EMB // L):
                                sbuf[gg, pl.ds(j * L, L)] = (
                                    sbuf[gg, pl.ds(j * L, L)] * bc)
                            return 0
                        lax.fori_loop(0, L, edge, 0)
                        return 0
                    lax.fori_loop(0, GBB // L, subgrp, 0)

                    hs[ps] = pltpu.async_copy(
                        sbuf, sh_acc.at[idx_d2.at[ib]], SS[ps], add=True)
                    if ib < NB - 2:
                        pn = (ib + 2) & 3
                        if hs[pn] is not None:
                            hs[pn].wait()
                        hg[pn] = issue_g(ib + 2, pn)
                for ps in range(4):
                    hs[ps].wait()
                return 0
            lax.fori_loop(0, TILE_B // BLK_B, block, 0)
            plsc.subcore_barrier()

            for q in range(nrows // GBB):
                pltpu.sync_copy(sh_acc.at[pl.ds(r0 + q * GBB, GBB)], sb0)
                pltpu.sync_copy(sb0, obl.at[pl.ds(r0 + q * GBB, GBB)])

        @pl.when(cid == 0)
        def _():
            chunk_pass(t0_hbm, o0_hbm, 0)
            chunk_pass(t1_hbm, o1_hbm, 1)

        @pl.when(cid == 1)
        def _():
            chunk_pass(t2_hbm, o2_hbm, 2)
            chunk_pass(t3_hbm, o3_hbm, 3)

    return k(src2, dst2, alpha, t0, t1, t2, t3)


def _k15_body(dp_ref, out_ref):
    out_ref[...] = jnp.sum(dp_ref[...], axis=0) + 1e-16


def _k15(denp):
    return pl.pallas_call(
        _k15_body,
        grid=(DMR // 64,),
        in_specs=[pl.BlockSpec((NC * NS, 64, DMC), lambda i: (0, i, 0))],
        out_specs=pl.BlockSpec((64, DMC), lambda i: (i, 0)),
        out_shape=jax.ShapeDtypeStruct((DMR, DMC), jnp.float32),
    )(denp.reshape(NC * NS, DMR, DMC))


def _k2a_body(c0, c1, c2, c3, bias, stats):
    i = pl.program_id(0)
    y = jnp.concatenate([c0[...], c1[...], c2[...], c3[...]], axis=1) + bias[...]
    blk = jnp.concatenate([jnp.sum(y, axis=0, keepdims=True),
                           jnp.sum(y * y, axis=0, keepdims=True)], axis=0)

    @pl.when(i == 0)
    def _():
        stats[...] = blk

    @pl.when(i > 0)
    def _():
        stats[...] = stats[...] + blk


def _k2b_body(c0, c1, c2, c3, bias, stats1, g1, b1, wlin, w2, b2_, w3, b3_,
              z_ref, stats2):
    i = pl.program_id(0)
    y = jnp.concatenate([c0[...], c1[...], c2[...], c3[...]], axis=1) + bias[...]
    mean = stats1[0:1, :] * (1.0 / N)
    var = stats1[1:2, :] * (1.0 / N) - mean * mean
    yn = (y - mean) * lax.rsqrt(var + EPS) * g1[...] + b1[...]
    x1 = jnp.dot(yn, wlin[...], preferred_element_type=jnp.float32)
    h = jnp.maximum(jnp.dot(x1, w2[...], preferred_element_type=jnp.float32)
                    + b2_[...], 0.0)
    hh = jnp.dot(h, w3[...], preferred_element_type=jnp.float32) + b3_[...]
    z = x1 + hh
    z_ref[...] = z
    blk = jnp.concatenate([jnp.sum(z, axis=0, keepdims=True),
                           jnp.sum(z * z, axis=0, keepdims=True)], axis=0)

    @pl.when(i == 0)
    def _():
        stats2[...] = blk

    @pl.when(i > 0)
    def _():
        stats2[...] = stats2[...] + blk


def _k2c_body(z, stats2, g2, b2_, out):
    mean = stats2[0:1, :] * (1.0 / N)
    var = stats2[1:2, :] * (1.0 / N) - mean * mean
    out[...] = (z[...] - mean) * lax.rsqrt(var + EPS) * g2[...] + b2_[...]


def kernel(node_attr, edge_index, Wl, bl, Wr, br, att, bias_gat, gamma1,
           beta1, W_lin, W2, b2, W3, b3, gamma2, beta2):
    x = jnp.zeros((NP, IN_CH), jnp.float32).at[:N].set(node_attr)
    loop = jnp.arange(N, dtype=jnp.int32)
    pad = jnp.full((EP - E - N,), N, jnp.int32)
    src = jnp.concatenate([edge_index[0], loop, pad])
    dst = jnp.concatenate([edge_index[1], loop, pad])
    att_de = att.reshape(HC // 32, 16, 2).transpose(0, 2, 1).reshape(HC)

    xl, xr, t0, t1, t2, t3 = _k1(x, Wl, bl.reshape(1, HC),
                                 Wr, br.reshape(1, HC))
    xlb = lax.bitcast_convert_type(
        xl.astype(jnp.bfloat16).reshape(NP, HC // 2, 2), jnp.int32)
    xrb = lax.bitcast_convert_type(
        xr.astype(jnp.bfloat16).reshape(NP, HC // 2, 2), jnp.int32)
    ex, denp = _pass_a(xlb, xrb, src, dst, att_de)

    denm = _k15(denp).reshape(DEN_W)
    alpha = _pass_a5(dst, ex, denm)
    o0, o1, o2, o3 = _pass_b(src.reshape(EP // GBB, GBB),
                             dst.reshape(EP // GBB, GBB), alpha,
                             t0, t1, t2, t3)

    rows = 1000
    stats1 = pl.pallas_call(
        _k2a_body,
        grid=(10,),
        in_specs=[pl.BlockSpec((rows, EMB), lambda i: (i, 0))] * 4
        + [pl.BlockSpec((1, HC), lambda i: (0, 0))],
        out_specs=pl.BlockSpec((2, HC), lambda i: (0, 0)),
        out_shape=jax.ShapeDtypeStruct((2, HC), jnp.float32),
    )(o0, o1, o2, o3, bias_gat.reshape(1, HC))

    z, stats2 = pl.pallas_call(
        _k2b_body,
        grid=(10,),
        in_specs=[pl.BlockSpec((rows, EMB), lambda i: (i, 0))] * 4
        + [pl.BlockSpec((1, HC), lambda i: (0, 0)),
           pl.BlockSpec((2, HC), lambda i: (0, 0)),
           pl.BlockSpec((1, HC), lambda i: (0, 0)),
           pl.BlockSpec((1, HC), lambda i: (0, 0)),
           pl.BlockSpec((HC, EMB), lambda i: (0, 0)),
           pl.BlockSpec((EMB, FF), lambda i: (0, 0)),
           pl.BlockSpec((1, FF), lambda i: (0, 0)),
           pl.BlockSpec((FF, EMB), lambda i: (0, 0)),
           pl.BlockSpec((1, EMB), lambda i: (0, 0))],
        out_specs=[pl.BlockSpec((rows, EMB), lambda i: (i, 0)),
                   pl.BlockSpec((2, EMB), lambda i: (0, 0))],
        out_shape=[jax.ShapeDtypeStruct((N, EMB), jnp.float32),
                   jax.ShapeDtypeStruct((2, EMB), jnp.float32)],
    )(o0, o1, o2, o3, bias_gat.reshape(1, HC), stats1, gamma1.reshape(1, HC),
      beta1.reshape(1, HC), W_lin, W2, b2.reshape(1, FF), W3,
      b3.reshape(1, EMB))

    out = pl.pallas_call(
        _k2c_body,
        grid=(10,),
        in_specs=[pl.BlockSpec((rows, EMB), lambda i: (i, 0)),
                  pl.BlockSpec((2, EMB), lambda i: (0, 0)),
                  pl.BlockSpec((1, EMB), lambda i: (0, 0)),
                  pl.BlockSpec((1, EMB), lambda i: (0, 0))],
        out_specs=pl.BlockSpec((rows, EMB), lambda i: (i, 0)),
        out_shape=jax.ShapeDtypeStruct((N, EMB), jnp.float32),
    )(z, stats2, gamma2.reshape(1, EMB), beta2.reshape(1, EMB))
    return out


# speedup vs baseline: 1.3813x; 1.1920x over previous
import functools

import jax
import jax.numpy as jnp
from jax import lax
from jax.experimental import pallas as pl
from jax.experimental.pallas import tpu as pltpu
from jax.experimental.pallas import tpu_sc as plsc

N = 10000
IN_CH = 256
EMB = 128
HEADS = 4
HC = HEADS * EMB
FF = 512
NEG = 0.2
EPS = 1e-5
E = 160000

NP = 10240
EP = 172032
NC, NS, L = 2, 16, 16
TILE_A = EP // (NC * NS)
TILE_B = EP // NS
GA = 128
GS = 64
GB = 128
NBA = TILE_A // GA
NBB = TILE_B // GB
DEN_W = NP * 4
DMR, DMC = DEN_W // 128, 128


def _dyn_gather16(v, idx):
    dnums = lax.GatherDimensionNumbers(
        offset_dims=(), collapsed_slice_dims=(0,), start_index_map=(0,))
    return lax.gather(v, idx[:, None], dnums, slice_sizes=(1,),
                      mode=lax.GatherScatterMode.PROMISE_IN_BOUNDS)


def _k1_body(x_ref, wl_ref, bl_ref, wr_ref, br_ref,
             xl_ref, xr_ref, xlb_ref, xrb_ref,
             c0_ref, c1_ref, c2_ref, c3_ref):
    x = x_ref[...]
    xl = jnp.dot(x, wl_ref[...], preferred_element_type=jnp.float32) + bl_ref[...]
    xr = jnp.dot(x, wr_ref[...], preferred_element_type=jnp.float32) + br_ref[...]
    xl_ref[...] = xl
    xr_ref[...] = xr
    xli = lax.bitcast_convert_type(
        xl.astype(jnp.bfloat16).astype(jnp.float32), jnp.int32)
    xri = lax.bitcast_convert_type(
        xr.astype(jnp.bfloat16).astype(jnp.float32), jnp.int32)
    xlb_ref[...] = lax.bitwise_or(
        lax.shift_right_logical(xli[:, 0:HC // 2], 16),
        lax.bitwise_and(xli[:, HC // 2:HC], -65536))
    xrb_ref[...] = lax.bitwise_or(
        lax.shift_right_logical(xri[:, 0:HC // 2], 16),
        lax.bitwise_and(xri[:, HC // 2:HC], -65536))
    c0_ref[...] = xl[:, 0:128]
    c1_ref[...] = xl[:, 128:256]
    c2_ref[...] = xl[:, 256:384]
    c3_ref[...] = xl[:, 384:512]


def _k1(x, Wl, bl2, Wr, br2):
    blk = NP // 8
    return pl.pallas_call(
        _k1_body,
        grid=(8,),
        in_specs=[
            pl.BlockSpec((blk, IN_CH), lambda i: (i, 0)),
            pl.BlockSpec((IN_CH, HC), lambda i: (0, 0)),
            pl.BlockSpec((1, HC), lambda i: (0, 0)),
            pl.BlockSpec((IN_CH, HC), lambda i: (0, 0)),
            pl.BlockSpec((1, HC), lambda i: (0, 0)),
        ],
        out_specs=[
            pl.BlockSpec((blk, HC), lambda i: (i, 0)),
            pl.BlockSpec((blk, HC), lambda i: (i, 0)),
            pl.BlockSpec((blk, HC // 2), lambda i: (i, 0)),
            pl.BlockSpec((blk, HC // 2), lambda i: (i, 0)),
            pl.BlockSpec((blk, EMB), lambda i: (i, 0)),
            pl.BlockSpec((blk, EMB), lambda i: (i, 0)),
            pl.BlockSpec((blk, EMB), lambda i: (i, 0)),
            pl.BlockSpec((blk, EMB), lambda i: (i, 0)),
        ],
        out_shape=[
            jax.ShapeDtypeStruct((NP, HC), jnp.float32),
            jax.ShapeDtypeStruct((NP, HC), jnp.float32),
            jax.ShapeDtypeStruct((NP, HC // 2), jnp.int32),
            jax.ShapeDtypeStruct((NP, HC // 2), jnp.int32),
            jax.ShapeDtypeStruct((NP, EMB), jnp.float32),
            jax.ShapeDtypeStruct((NP, EMB), jnp.float32),
            jax.ShapeDtypeStruct((NP, EMB), jnp.float32),
            jax.ShapeDtypeStruct((NP, EMB), jnp.float32),
        ],
    )(x, Wl, bl2, Wr, br2)


def _pass_a(xl, xr, src, dst, att_flat):
    mesh = plsc.VectorSubcoreMesh(core_axis_name="c", subcore_axis_name="s")
    HW = HC // 2

    @functools.partial(
        pl.kernel,
        mesh=mesh,
        compiler_params=pltpu.CompilerParams(needs_layout_passes=False),
        out_type=[
            jax.ShapeDtypeStruct((4, EP), jnp.float32),
            jax.ShapeDtypeStruct((NC * NS * DEN_W,), jnp.float32),
        ],
        scratch_types=[
            pltpu.VMEM((128,), jnp.int32),
            pltpu.VMEM((128,), jnp.int32),
            pltpu.VMEM((128,), jnp.int32),
            pltpu.VMEM((128,), jnp.int32),
            pltpu.VMEM((GS, HW), jnp.int32),
            pltpu.VMEM((GS, HW), jnp.int32),
            pltpu.VMEM((GS, HW), jnp.int32),
            pltpu.VMEM((GS, HW), jnp.int32),
            pltpu.VMEM((4, 256), jnp.float32),
            pltpu.VMEM((HC,), jnp.float32),
            pltpu.VMEM((DEN_W,), jnp.float32),
            pltpu.SemaphoreType.DMA,
            pltpu.SemaphoreType.DMA,
            pltpu.SemaphoreType.DMA,
            pltpu.SemaphoreType.DMA,
            pltpu.SemaphoreType.DMA,
            pltpu.SemaphoreType.DMA,
            pltpu.SemaphoreType.DMA,
            pltpu.SemaphoreType.DMA,
        ],
    )
    def k(xl_hbm, xr_hbm, src_hbm, dst_hbm, att_hbm, ex_hbm, den_hbm,
          ids0, ids1, idd0, idd1, rl0, rl1, rr0, rr1, ex_buf, att_v, den_v,
          sis0, sis1, sid0, sid1, sl0, sl1, sr0, sr1):
        cid = lax.axis_index("c")
        sid = lax.axis_index("s")
        tid = sid * NC + cid
        ii = lax.iota(jnp.int32, L)
        zv = jnp.zeros((L,), jnp.float32)
        IDS, IDD = (ids0, ids1), (idd0, idd1)
        SIS, SID = (sis0, sis1), (sid0, sid1)
        RL, RR = (rl0, rl1), (rr0, rr1)
        SL, SR = (sl0, sl1), (sr0, sr1)

        def zb(i, _):
            den_v[pl.ds(lax.mul(i, L), L)] = zv
            return 0
        lax.fori_loop(0, DEN_W // L, zb, 0)

        pltpu.sync_copy(att_hbm, att_v)
        att_sl = [att_v[pl.ds(w * L, L)] for w in range(HC // L)]

        ebase = lax.mul(tid, TILE_A)

        def issue_idx(jq, p):
            off = jnp.minimum(ebase + jq * 128, EP - 128)
            pltpu.async_copy(src_hbm.at[pl.ds(off, 128)], IDS[p], SIS[p])
            pltpu.async_copy(dst_hbm.at[pl.ds(off, 128)], IDD[p], SID[p])

        def wait_idx(p):
            pltpu.make_async_copy(
                src_hbm.at[pl.ds(0, 128)], IDS[p], SIS[p]).wait()
            pltpu.make_async_copy(
                dst_hbm.at[pl.ds(0, 128)], IDD[p], SID[p]).wait()

        def issue_g(half, b, pidx):
            pltpu.async_copy(
                xl_hbm.at[IDS[pidx].at[pl.ds(half * GS, GS)]], RL[b], SL[b])
            pltpu.async_copy(
                xr_hbm.at[IDD[pidx].at[pl.ds(half * GS, GS)]], RR[b], SR[b])

        def wait_g(b):
            pltpu.make_async_copy(
                xl_hbm.at[pl.ds(0, GS)], RL[b], SL[b]).wait()
            pltpu.make_async_copy(
                xr_hbm.at[pl.ds(0, GS)], RR[b], SR[b]).wait()

        def compute(half, b, pidx, exoff):
            rl, rr = RL[b], RR[b]
            idd = IDD[pidx]

            def subgrp(sg, _):
                colq = half * GS + sg * L
                col = exoff + colq

                def edge(g, lv):
                    gg = sg * L + g
                    sel = ii == g
                    accs = [jnp.zeros((L,), jnp.float32)
                            for _ in range(HEADS)]
                    for w in range(16):
                        wl = rl[gg, pl.ds(w * L, L)]
                        wr = rr[gg, pl.ds(w * L, L)]
                        le = plsc.bitcast(
                            lax.shift_left(wl, 16), jnp.float32)
                        he = plsc.bitcast(
                            lax.bitwise_and(wl, -65536), jnp.float32)
                        re_ = plsc.bitcast(
                            lax.shift_left(wr, 16), jnp.float32)
                        hr = plsc.bitcast(
                            lax.bitwise_and(wr, -65536), jnp.float32)
                        tl = le + re_
                        tl = jnp.maximum(tl, NEG * tl)
                        accs[w // 8] = accs[w // 8] + tl * att_sl[w]
                        th = he + hr
                        th = jnp.maximum(th, NEG * th)
                        accs[2 + w // 8] = (accs[2 + w // 8]
                                            + th * att_sl[16 + w])
                    for h in range(HEADS):
                        red = accs[h]
                        for st in (8, 4, 2, 1):
                            red = red + _dyn_gather16(
                                red, lax.bitwise_xor(ii, st))
                        lv = (lv[:h] + (jnp.where(sel, red, lv[h]),)
                              + lv[h + 1:])
                    return lv
                lv = lax.fori_loop(0, L, edge, (zv, zv, zv, zv))
                dv = idd[pl.ds(colq, L)]
                for h in range(HEADS):
                    ev = jnp.exp(lv[h])
                    ex_buf[h, pl.ds(col, L)] = ev
                    plsc.addupdate_scatter(den_v, [dv * 4 + h], ev)
                return 0
            lax.fori_loop(0, GS // L, subgrp, 0)

        issue_idx(0, 0)
        issue_idx(1, 1)
        wait_idx(0)
        issue_g(0, 0, 0)
        issue_g(1, 1, 0)

        def pair(m, _):
            a2 = m * 2
            wait_g(0)
            compute(0, 0, 0, 0)
            wait_idx(1)
            issue_g(0, 0, 1)
            wait_g(1)
            compute(1, 1, 0, 0)
            issue_g(1, 1, 1)
            issue_idx(a2 + 2, 0)
            wait_g(0)
            compute(0, 0, 1, 128)
            wait_idx(0)
            issue_g(0, 0, 0)
            wait_g(1)
            compute(1, 1, 1, 128)
            issue_g(1, 1, 0)
            issue_idx(a2 + 3, 1)
            eb = ebase + m * 256
            for h in range(HEADS):
                pltpu.sync_copy(ex_buf.at[h],
                                ex_hbm.at[h].at[pl.ds(eb, 256)])
            return 0
        lax.fori_loop(0, TILE_A // 256, pair, 0)
        wait_g(0)
        wait_g(1)
        wait_idx(1)

        pltpu.sync_copy(den_v,
                        den_hbm.at[pl.ds(lax.mul(tid, DEN_W), DEN_W)])

    return k(xl, xr, src, dst, att_flat)


A5B = 384


def _pass_a5(dst, ex, denm):
    mesh = plsc.VectorSubcoreMesh(core_axis_name="c", subcore_axis_name="s")

    @functools.partial(
        pl.kernel,
        mesh=mesh,
        compiler_params=pltpu.CompilerParams(needs_layout_passes=False),
        out_type=jax.ShapeDtypeStruct((4, EP), jnp.float32),
        scratch_types=[
            pltpu.VMEM((A5B,), jnp.int32),
            pltpu.VMEM((4, A5B), jnp.float32),
            pltpu.VMEM((4, A5B), jnp.float32),
            pltpu.VMEM((DEN_W,), jnp.float32),
        ],
    )
    def k(dst_hbm, ex_hbm, den_hbm, al_hbm, idx_d, exb, alb, d_v):
        cid = lax.axis_index("c")
        sid = lax.axis_index("s")
        tid = sid * NC + cid
        ii = lax.iota(jnp.int32, L)
        pltpu.sync_copy(den_hbm, d_v)
        ebase = lax.mul(tid, TILE_A)

        def batch(b, _):
            eb = ebase + b * A5B
            pltpu.sync_copy(dst_hbm.at[pl.ds(eb, A5B)], idx_d)
            for h in range(HEADS):
                pltpu.sync_copy(ex_hbm.at[h].at[pl.ds(eb, A5B)],
                                exb.at[h].at[pl.ds(0, A5B)])
            for sub in range(0, A5B, L):
                dv = idx_d[pl.ds(sub, L)]
                for h in range(HEADS):
                    fi = dv * 4 + h
                    den = plsc.load_gather(d_v, [fi])
                    alb[h, pl.ds(sub, L)] = exb[h, pl.ds(sub, L)] / den
            for h in range(HEADS):
                pltpu.sync_copy(alb.at[h].at[pl.ds(0, A5B)],
                                al_hbm.at[h].at[pl.ds(eb, A5B)])
            return 0
        lax.fori_loop(0, TILE_A // A5B, batch, 0)

    return k(dst, ex, denm)


GBB = 64
BLK_B = 1536


def _pass_b(src2, dst2, alpha, t0, t1, t2, t3):
    mesh = plsc.VectorSubcoreMesh(core_axis_name="c", subcore_axis_name="s")
    CW = EMB // 2

    @functools.partial(
        pl.kernel,
        mesh=mesh,
        compiler_params=pltpu.CompilerParams(needs_layout_passes=False),
        out_type=[jax.ShapeDtypeStruct((NP, EMB), jnp.float32)
                  for _ in range(4)],
        scratch_types=[
            pltpu.VMEM((BLK_B // GBB, GBB), jnp.int32),
            pltpu.VMEM((BLK_B // GBB, GBB), jnp.int32),
            pltpu.VMEM((BLK_B,), jnp.float32),
            pltpu.VMEM((GBB, EMB), jnp.float32),
            pltpu.VMEM((GBB, EMB), jnp.float32),
            pltpu.VMEM((GBB, EMB), jnp.float32),
            pltpu.VMEM((GBB, EMB), jnp.float32),
            pltpu.VMEM_SHARED((NP, EMB), jnp.float32),
            pltpu.SemaphoreType.DMA,
            pltpu.SemaphoreType.DMA,
            pltpu.SemaphoreType.DMA,
            pltpu.SemaphoreType.DMA,
            pltpu.SemaphoreType.DMA,
            pltpu.SemaphoreType.DMA,
            pltpu.SemaphoreType.DMA,
            pltpu.SemaphoreType.DMA,
        ],
    )
    def k(src2_hbm, dst2_hbm, al_hbm, t0_hbm, t1_hbm, t2_hbm, t3_hbm,
          o0_hbm, o1_hbm, o2_hbm, o3_hbm,
          idx_s2, idx_d2, alb, sb0, sb1, sb2, sb3, sh_acc,
          sg0, sg1, sg2, sg3, ss0, ss1, ss2, ss3):
        cid = lax.axis_index("c")
        sid = lax.axis_index("s")
        ii = lax.iota(jnp.int32, L)
        zv = jnp.zeros((L,), jnp.float32)
        SB = (sb0, sb1, sb2, sb3)
        SG = (sg0, sg1, sg2, sg3)
        SS = (ss0, ss1, ss2, ss3)
        NB = BLK_B // GBB

        nrows = NP // NS
        r0 = lax.mul(sid, nrows)
        ebase = lax.mul(sid, TILE_B)
        rbase = lax.mul(sid, TILE_B // GBB)

        def chunk_pass(tbl, obl, c):
            def zr(i, _):
                sb0[lax.shift_right_logical(i, 3),
                    pl.ds(lax.mul(lax.bitwise_and(i, 7), L), L)] = zv
                return 0
            lax.fori_loop(0, GBB * (EMB // L), zr, 0)
            for q in range(nrows // GBB):
                pltpu.sync_copy(sb0, sh_acc.at[pl.ds(r0 + q * GBB, GBB)])
            plsc.subcore_barrier()

            def issue_g(ib, p):
                return pltpu.async_copy(
                    tbl.at[idx_s2.at[ib]], SB[p], SG[p])

            def block(b, _):
                eb = ebase + b * BLK_B
                rr = rbase + b * NB
                pltpu.sync_copy(src2_hbm.at[pl.ds(rr, NB)], idx_s2)
                pltpu.sync_copy(dst2_hbm.at[pl.ds(rr, NB)], idx_d2)
                pltpu.sync_copy(al_hbm.at[c].at[pl.ds(eb, BLK_B)], alb)
                hg = [issue_g(0, 0), issue_g(1, 1), None, None]
                hs = [None, None, None, None]
                for ib in range(NB):
                    ps = ib & 3
                    hg[ps].wait()
                    sbuf = SB[ps]

                    def subgrp(sg_, _):
                        base16 = lax.mul(sg_, L)
                        av = alb[pl.ds(ib * GBB + base16, L)]

                        def edge(g, _):
                            gg = base16 + g
                            bc = _dyn_gather16(
                                av, jnp.full((L,), g, jnp.int32))
                            for j in range(EMB // L):
                                sbuf[gg, pl.ds(j * L, L)] = (
                                    sbuf[gg, pl.ds(j * L, L)] * bc)
                            return 0
                        lax.fori_loop(0, L, edge, 0)
                        return 0
                    lax.fori_loop(0, GBB // L, subgrp, 0)

                    hs[ps] = pltpu.async_copy(
                        sbuf, sh_acc.at[idx_d2.at[ib]], SS[ps], add=True)
                    if ib < NB - 2:
                        pn = (ib + 2) & 3
                        if hs[pn] is not None:
                            hs[pn].wait()
                        hg[pn] = issue_g(ib + 2, pn)
                for ps in range(4):
                    hs[ps].wait()
                return 0
            lax.fori_loop(0, TILE_B // BLK_B, block, 0)
            plsc.subcore_barrier()

            for q in range(nrows // GBB):
                pltpu.sync_copy(sh_acc.at[pl.ds(r0 + q * GBB, GBB)], sb0)
                pltpu.sync_copy(sb0, obl.at[pl.ds(r0 + q * GBB, GBB)])

        @pl.when(cid == 0)
        def _():
            chunk_pass(t0_hbm, o0_hbm, 0)
            chunk_pass(t1_hbm, o1_hbm, 1)

        @pl.when(cid == 1)
        def _():
            chunk_pass(t2_hbm, o2_hbm, 2)
            chunk_pass(t3_hbm, o3_hbm, 3)

    return k(src2, dst2, alpha, t0, t1, t2, t3)


def _k15_body(dp_ref, out_ref):
    out_ref[...] = jnp.sum(dp_ref[...], axis=0) + 1e-16


def _k15(denp):
    return pl.pallas_call(
        _k15_body,
        grid=(DMR // 64,),
        in_specs=[pl.BlockSpec((NC * NS, 64, DMC), lambda i: (0, i, 0))],
        out_specs=pl.BlockSpec((64, DMC), lambda i: (i, 0)),
        out_shape=jax.ShapeDtypeStruct((DMR, DMC), jnp.float32),
    )(denp.reshape(NC * NS, DMR, DMC))


def _k2a_body(c0, c1, c2, c3, bias, stats):
    i = pl.program_id(0)
    y = jnp.concatenate([c0[...], c1[...], c2[...], c3[...]], axis=1) + bias[...]
    blk = jnp.concatenate([jnp.sum(y, axis=0, keepdims=True),
                           jnp.sum(y * y, axis=0, keepdims=True)], axis=0)

    @pl.when(i == 0)
    def _():
        stats[...] = blk

    @pl.when(i > 0)
    def _():
        stats[...] = stats[...] + blk


def _k2b_body(c0, c1, c2, c3, bias, stats1, g1, b1, wlin, w2, b2_, w3, b3_,
              z_ref, stats2):
    i = pl.program_id(0)
    y = jnp.concatenate([c0[...], c1[...], c2[...], c3[...]], axis=1) + bias[...]
    mean = stats1[0:1, :] * (1.0 / N)
    var = stats1[1:2, :] * (1.0 / N) - mean * mean
    yn = (y - mean) * lax.rsqrt(var + EPS) * g1[...] + b1[...]
    x1 = jnp.dot(yn, wlin[...], preferred_element_type=jnp.float32)
    h = jnp.maximum(jnp.dot(x1, w2[...], preferred_element_type=jnp.float32)
                    + b2_[...], 0.0)
    hh = jnp.dot(h, w3[...], preferred_element_type=jnp.float32) + b3_[...]
    z = x1 + hh
    z_ref[...] = z
    blk = jnp.concatenate([jnp.sum(z, axis=0, keepdims=True),
                           jnp.sum(z * z, axis=0, keepdims=True)], axis=0)

    @pl.when(i == 0)
    def _():
        stats2[...] = blk

    @pl.when(i > 0)
    def _():
        stats2[...] = stats2[...] + blk


def _k2c_body(z, stats2, g2, b2_, out):
    mean = stats2[0:1, :] * (1.0 / N)
    var = stats2[1:2, :] * (1.0 / N) - mean * mean
    out[...] = (z[...] - mean) * lax.rsqrt(var + EPS) * g2[...] + b2_[...]


def kernel(node_attr, edge_index, Wl, bl, Wr, br, att, bias_gat, gamma1,
           beta1, W_lin, W2, b2, W3, b3, gamma2, beta2):
    x = jnp.zeros((NP, IN_CH), jnp.float32).at[:N].set(node_attr)
    loop = jnp.arange(N, dtype=jnp.int32)
    pad = jnp.full((EP - E - N,), N, jnp.int32)
    src = jnp.concatenate([edge_index[0], loop, pad])
    dst = jnp.concatenate([edge_index[1], loop, pad])
    att_flat = att.reshape(HC)

    (xl, xr, xlb, xrb, t0, t1, t2, t3) = _k1(
        x, Wl, bl.reshape(1, HC), Wr, br.reshape(1, HC))
    ex, denp = _pass_a(xlb, xrb, src, dst, att_flat)

    denm = _k15(denp).reshape(DEN_W)
    alpha = _pass_a5(dst, ex, denm)
    o0, o1, o2, o3 = _pass_b(src.reshape(EP // GBB, GBB),
                             dst.reshape(EP // GBB, GBB), alpha,
                             t0, t1, t2, t3)

    rows = 1000
    stats1 = pl.pallas_call(
        _k2a_body,
        grid=(10,),
        in_specs=[pl.BlockSpec((rows, EMB), lambda i: (i, 0))] * 4
        + [pl.BlockSpec((1, HC), lambda i: (0, 0))],
        out_specs=pl.BlockSpec((2, HC), lambda i: (0, 0)),
        out_shape=jax.ShapeDtypeStruct((2, HC), jnp.float32),
    )(o0, o1, o2, o3, bias_gat.reshape(1, HC))

    z, stats2 = pl.pallas_call(
        _k2b_body,
        grid=(10,),
        in_specs=[pl.BlockSpec((rows, EMB), lambda i: (i, 0))] * 4
        + [pl.BlockSpec((1, HC), lambda i: (0, 0)),
           pl.BlockSpec((2, HC), lambda i: (0, 0)),
           pl.BlockSpec((1, HC), lambda i: (0, 0)),
           pl.BlockSpec((1, HC), lambda i: (0, 0)),
           pl.BlockSpec((HC, EMB), lambda i: (0, 0)),
           pl.BlockSpec((EMB, FF), lambda i: (0, 0)),
           pl.BlockSpec((1, FF), lambda i: (0, 0)),
           pl.BlockSpec((FF, EMB), lambda i: (0, 0)),
           pl.BlockSpec((1, EMB), lambda i: (0, 0))],
        out_specs=[pl.BlockSpec((rows, EMB), lambda i: (i, 0)),
                   pl.BlockSpec((2, EMB), lambda i: (0, 0))],
        out_shape=[jax.ShapeDtypeStruct((N, EMB), jnp.float32),
                   jax.ShapeDtypeStruct((2, EMB), jnp.float32)],
    )(o0, o1, o2, o3, bias_gat.reshape(1, HC), stats1, gamma1.reshape(1, HC),
      beta1.reshape(1, HC), W_lin, W2, b2.reshape(1, FF), W3,
      b3.reshape(1, EMB))

    out = pl.pallas_call(
        _k2c_body,
        grid=(10,),
        in_specs=[pl.BlockSpec((rows, EMB), lambda i: (i, 0)),
                  pl.BlockSpec((2, EMB), lambda i: (0, 0)),
                  pl.BlockSpec((1, EMB), lambda i: (0, 0)),
                  pl.BlockSpec((1, EMB), lambda i: (0, 0))],
        out_specs=pl.BlockSpec((rows, EMB), lambda i: (i, 0)),
        out_shape=jax.ShapeDtypeStruct((N, EMB), jnp.float32),
    )(z, stats2, gamma2.reshape(1, EMB), beta2.reshape(1, EMB))
    return out


# speedup vs baseline: 1.4610x; 1.0577x over previous
import functools

import jax
import jax.numpy as jnp
from jax import lax
from jax.experimental import pallas as pl
from jax.experimental.pallas import tpu as pltpu
from jax.experimental.pallas import tpu_sc as plsc

N = 10000
IN_CH = 256
EMB = 128
HEADS = 4
HC = HEADS * EMB
FF = 512
NEG = 0.2
EPS = 1e-5
E = 160000

NP = 10240
EP = 172032
NC, NS, L = 2, 16, 16
TILE_A = EP // (NC * NS)
TILE_B = EP // NS
GA = 128
GS = 64
GB = 128
NBA = TILE_A // GA
NBB = TILE_B // GB
DEN_W = NP * 4
DMR, DMC = DEN_W // 128, 128


def _dyn_gather16(v, idx):
    dnums = lax.GatherDimensionNumbers(
        offset_dims=(), collapsed_slice_dims=(0,), start_index_map=(0,))
    return lax.gather(v, idx[:, None], dnums, slice_sizes=(1,),
                      mode=lax.GatherScatterMode.PROMISE_IN_BOUNDS)


def _k1_body(x_ref, wl_ref, bl_ref, wr_ref, br_ref,
             xl_ref, xr_ref, xlb_ref, xrb_ref,
             c0_ref, c1_ref, c2_ref, c3_ref):
    x = x_ref[...]
    xl = jnp.dot(x, wl_ref[...], preferred_element_type=jnp.float32) + bl_ref[...]
    xr = jnp.dot(x, wr_ref[...], preferred_element_type=jnp.float32) + br_ref[...]
    xl_ref[...] = xl
    xr_ref[...] = xr
    xli = lax.bitcast_convert_type(
        xl.astype(jnp.bfloat16).astype(jnp.float32), jnp.int32)
    xri = lax.bitcast_convert_type(
        xr.astype(jnp.bfloat16).astype(jnp.float32), jnp.int32)
    xlb_ref[...] = lax.bitwise_or(
        lax.shift_right_logical(xli[:, 0:HC // 2], 16),
        lax.bitwise_and(xli[:, HC // 2:HC], -65536))
    xrb_ref[...] = lax.bitwise_or(
        lax.shift_right_logical(xri[:, 0:HC // 2], 16),
        lax.bitwise_and(xri[:, HC // 2:HC], -65536))
    c0_ref[...] = xl[:, 0:128]
    c1_ref[...] = xl[:, 128:256]
    c2_ref[...] = xl[:, 256:384]
    c3_ref[...] = xl[:, 384:512]


def _k1(x, Wl, bl2, Wr, br2):
    blk = NP // 8
    return pl.pallas_call(
        _k1_body,
        grid=(8,),
        in_specs=[
            pl.BlockSpec((blk, IN_CH), lambda i: (i, 0)),
            pl.BlockSpec((IN_CH, HC), lambda i: (0, 0)),
            pl.BlockSpec((1, HC), lambda i: (0, 0)),
            pl.BlockSpec((IN_CH, HC), lambda i: (0, 0)),
            pl.BlockSpec((1, HC), lambda i: (0, 0)),
        ],
        out_specs=[
            pl.BlockSpec((blk, HC), lambda i: (i, 0)),
            pl.BlockSpec((blk, HC), lambda i: (i, 0)),
            pl.BlockSpec((blk, HC // 2), lambda i: (i, 0)),
            pl.BlockSpec((blk, HC // 2), lambda i: (i, 0)),
            pl.BlockSpec((blk, EMB), lambda i: (i, 0)),
            pl.BlockSpec((blk, EMB), lambda i: (i, 0)),
            pl.BlockSpec((blk, EMB), lambda i: (i, 0)),
            pl.BlockSpec((blk, EMB), lambda i: (i, 0)),
        ],
        out_shape=[
            jax.ShapeDtypeStruct((NP, HC), jnp.float32),
            jax.ShapeDtypeStruct((NP, HC), jnp.float32),
            jax.ShapeDtypeStruct((NP, HC // 2), jnp.int32),
            jax.ShapeDtypeStruct((NP, HC // 2), jnp.int32),
            jax.ShapeDtypeStruct((NP, EMB), jnp.float32),
            jax.ShapeDtypeStruct((NP, EMB), jnp.float32),
            jax.ShapeDtypeStruct((NP, EMB), jnp.float32),
            jax.ShapeDtypeStruct((NP, EMB), jnp.float32),
        ],
    )(x, Wl, bl2, Wr, br2)


def _pass_a(xl, xr, src, dst, att_flat):
    mesh = plsc.VectorSubcoreMesh(core_axis_name="c", subcore_axis_name="s")
    HW = HC // 2

    @functools.partial(
        pl.kernel,
        mesh=mesh,
        compiler_params=pltpu.CompilerParams(needs_layout_passes=False),
        out_type=[
            jax.ShapeDtypeStruct((4, EP), jnp.float32),
            jax.ShapeDtypeStruct((NC * NS * DEN_W,), jnp.float32),
        ],
        scratch_types=[
            pltpu.VMEM((128,), jnp.int32),
            pltpu.VMEM((128,), jnp.int32),
            pltpu.VMEM((128,), jnp.int32),
            pltpu.VMEM((128,), jnp.int32),
            pltpu.VMEM((GS, HW), jnp.int32),
            pltpu.VMEM((GS, HW), jnp.int32),
            pltpu.VMEM((GS, HW), jnp.int32),
            pltpu.VMEM((GS, HW), jnp.int32),
            pltpu.VMEM((4, 256), jnp.float32),
            pltpu.VMEM((HC,), jnp.float32),
            pltpu.VMEM((DEN_W,), jnp.float32),
            pltpu.SemaphoreType.DMA,
            pltpu.SemaphoreType.DMA,
            pltpu.SemaphoreType.DMA,
            pltpu.SemaphoreType.DMA,
            pltpu.SemaphoreType.DMA,
            pltpu.SemaphoreType.DMA,
            pltpu.SemaphoreType.DMA,
            pltpu.SemaphoreType.DMA,
        ],
    )
    def k(xl_hbm, xr_hbm, src_hbm, dst_hbm, att_hbm, ex_hbm, den_hbm,
          ids0, ids1, idd0, idd1, rl0, rl1, rr0, rr1, ex_buf, att_v, den_v,
          sis0, sis1, sid0, sid1, sl0, sl1, sr0, sr1):
        cid = lax.axis_index("c")
        sid = lax.axis_index("s")
        tid = sid * NC + cid
        ii = lax.iota(jnp.int32, L)
        zv = jnp.zeros((L,), jnp.float32)
        IDS, IDD = (ids0, ids1), (idd0, idd1)
        SIS, SID = (sis0, sis1), (sid0, sid1)
        RL, RR = (rl0, rl1), (rr0, rr1)
        SL, SR = (sl0, sl1), (sr0, sr1)

        def zb(i, _):
            den_v[pl.ds(lax.mul(i, L), L)] = zv
            return 0
        lax.fori_loop(0, DEN_W // L, zb, 0)

        pltpu.sync_copy(att_hbm, att_v)
        att_sl = [att_v[pl.ds(w * L, L)] for w in range(HC // L)]

        ebase = lax.mul(tid, TILE_A)

        def issue_idx(jq, p):
            off = jnp.minimum(ebase + jq * 128, EP - 128)
            pltpu.async_copy(src_hbm.at[pl.ds(off, 128)], IDS[p], SIS[p])
            pltpu.async_copy(dst_hbm.at[pl.ds(off, 128)], IDD[p], SID[p])

        def wait_idx(p):
            pltpu.make_async_copy(
                src_hbm.at[pl.ds(0, 128)], IDS[p], SIS[p]).wait()
            pltpu.make_async_copy(
                dst_hbm.at[pl.ds(0, 128)], IDD[p], SID[p]).wait()

        def issue_g(half, b, pidx):
            pltpu.async_copy(
                xl_hbm.at[IDS[pidx].at[pl.ds(half * GS, GS)]], RL[b], SL[b])
            pltpu.async_copy(
                xr_hbm.at[IDD[pidx].at[pl.ds(half * GS, GS)]], RR[b], SR[b])

        def wait_g(b):
            pltpu.make_async_copy(
                xl_hbm.at[pl.ds(0, GS)], RL[b], SL[b]).wait()
            pltpu.make_async_copy(
                xr_hbm.at[pl.ds(0, GS)], RR[b], SR[b]).wait()

        def compute(half, b, pidx, exoff):
            rl, rr = RL[b], RR[b]
            idd = IDD[pidx]

            def subgrp(sg, _):
                colq = half * GS + sg * L
                col = exoff + colq

                def edge(g, lv):
                    gg = sg * L + g
                    sel = ii == g
                    accs = [jnp.zeros((L,), jnp.float32)
                            for _ in range(HEADS)]
                    for w in range(16):
                        wl = rl[gg, pl.ds(w * L, L)]
                        wr = rr[gg, pl.ds(w * L, L)]
                        le = plsc.bitcast(
                            lax.shift_left(wl, 16), jnp.float32)
                        he = plsc.bitcast(
                            lax.bitwise_and(wl, -65536), jnp.float32)
                        re_ = plsc.bitcast(
                            lax.shift_left(wr, 16), jnp.float32)
                        hr = plsc.bitcast(
                            lax.bitwise_and(wr, -65536), jnp.float32)
                        tl = le + re_
                        tl = jnp.maximum(tl, NEG * tl)
                        accs[w // 8] = accs[w // 8] + tl * att_sl[w]
                        th = he + hr
                        th = jnp.maximum(th, NEG * th)
                        accs[2 + w // 8] = (accs[2 + w // 8]
                                            + th * att_sl[16 + w])
                    for h in range(HEADS):
                        red = accs[h]
                        for st in (8, 4, 2, 1):
                            red = red + _dyn_gather16(
                                red, lax.bitwise_xor(ii, st))
                        lv = (lv[:h] + (jnp.where(sel, red, lv[h]),)
                              + lv[h + 1:])
                    return lv
                lv = lax.fori_loop(0, L, edge, (zv, zv, zv, zv))
                dv = idd[pl.ds(colq, L)]
                for h in range(HEADS):
                    ev = jnp.exp(lv[h])
                    ex_buf[h, pl.ds(col, L)] = ev
                    plsc.addupdate_scatter(den_v, [dv + h * NP], ev)
                return 0
            lax.fori_loop(0, GS // L, subgrp, 0)

        issue_idx(0, 0)
        issue_idx(1, 1)
        wait_idx(0)
        issue_g(0, 0, 0)
        issue_g(1, 1, 0)

        def pair(m, _):
            a2 = m * 2
            wait_g(0)
            compute(0, 0, 0, 0)
            wait_idx(1)
            issue_g(0, 0, 1)
            wait_g(1)
            compute(1, 1, 0, 0)
            issue_g(1, 1, 1)
            issue_idx(a2 + 2, 0)
            wait_g(0)
            compute(0, 0, 1, 128)
            wait_idx(0)
            issue_g(0, 0, 0)
            wait_g(1)
            compute(1, 1, 1, 128)
            issue_g(1, 1, 0)
            issue_idx(a2 + 3, 1)
            eb = ebase + m * 256
            for h in range(HEADS):
                pltpu.sync_copy(ex_buf.at[h],
                                ex_hbm.at[h].at[pl.ds(eb, 256)])
            return 0
        lax.fori_loop(0, TILE_A // 256, pair, 0)
        wait_g(0)
        wait_g(1)
        wait_idx(1)

        pltpu.sync_copy(den_v,
                        den_hbm.at[pl.ds(lax.mul(tid, DEN_W), DEN_W)])

    return k(xl, xr, src, dst, att_flat)


GBB = 48
BLK_B = 1536


def _pass_b(src2, dst2, ex, den, t0, t1, t2, t3):
    mesh = plsc.VectorSubcoreMesh(core_axis_name="c", subcore_axis_name="s")
    CW = EMB // 2

    @functools.partial(
        pl.kernel,
        mesh=mesh,
        compiler_params=pltpu.CompilerParams(needs_layout_passes=False),
        out_type=[jax.ShapeDtypeStruct((NP, EMB), jnp.float32)
                  for _ in range(4)],
        scratch_types=[
            pltpu.VMEM((BLK_B // GBB, GBB), jnp.int32),
            pltpu.VMEM((BLK_B // GBB, GBB), jnp.int32),
            pltpu.VMEM((BLK_B,), jnp.float32),
            pltpu.VMEM((NP,), jnp.float32),
            pltpu.VMEM((GBB, EMB), jnp.float32),
            pltpu.VMEM((GBB, EMB), jnp.float32),
            pltpu.VMEM((GBB, EMB), jnp.float32),
            pltpu.VMEM((GBB, EMB), jnp.float32),
            pltpu.VMEM_SHARED((NP, EMB), jnp.float32),
            pltpu.SemaphoreType.DMA,
            pltpu.SemaphoreType.DMA,
            pltpu.SemaphoreType.DMA,
            pltpu.SemaphoreType.DMA,
            pltpu.SemaphoreType.DMA,
            pltpu.SemaphoreType.DMA,
            pltpu.SemaphoreType.DMA,
            pltpu.SemaphoreType.DMA,
        ],
    )
    def k(src2_hbm, dst2_hbm, ex_hbm, den_hbm, t0_hbm, t1_hbm, t2_hbm,
          t3_hbm, o0_hbm, o1_hbm, o2_hbm, o3_hbm,
          idx_s2, idx_d2, alb, d_v, sb0, sb1, sb2, sb3, sh_acc,
          sg0, sg1, sg2, sg3, ss0, ss1, ss2, ss3):
        cid = lax.axis_index("c")
        sid = lax.axis_index("s")
        ii = lax.iota(jnp.int32, L)
        zv = jnp.zeros((L,), jnp.float32)
        SB = (sb0, sb1, sb2, sb3)
        SG = (sg0, sg1, sg2, sg3)
        SS = (ss0, ss1, ss2, ss3)
        NB = BLK_B // GBB

        nrows = NP // NS
        r0 = lax.mul(sid, nrows)
        ebase = lax.mul(sid, TILE_B)
        rbase = lax.mul(sid, TILE_B // GBB)

        def chunk_pass(tbl, obl, c):
            pltpu.sync_copy(den_hbm.at[pl.ds(c * NP, NP)], d_v)
            def zr(i, _):
                sb0[lax.shift_right_logical(i, 3),
                    pl.ds(lax.mul(lax.bitwise_and(i, 7), L), L)] = zv
                return 0
            lax.fori_loop(0, GBB * (EMB // L), zr, 0)
            for q in range(nrows // 40):
                pltpu.sync_copy(sb0.at[pl.ds(0, 40)],
                                sh_acc.at[pl.ds(r0 + q * 40, 40)])
            plsc.subcore_barrier()

            def issue_g(ib, p):
                return pltpu.async_copy(
                    tbl.at[idx_s2.at[ib]], SB[p], SG[p])

            def block(b, _):
                eb = ebase + b * BLK_B
                rr = rbase + b * NB
                pltpu.sync_copy(src2_hbm.at[pl.ds(rr, NB)], idx_s2)
                pltpu.sync_copy(dst2_hbm.at[pl.ds(rr, NB)], idx_d2)
                pltpu.sync_copy(ex_hbm.at[c].at[pl.ds(eb, BLK_B)], alb)
                hg = [issue_g(0, 0), issue_g(1, 1), None, None]
                hs = [None, None, None, None]
                for ib in range(NB):
                    ps = ib & 3
                    hg[ps].wait()
                    sbuf = SB[ps]

                    def subgrp(sg_, _):
                        base16 = lax.mul(sg_, L)
                        dv = idx_d2[ib, pl.ds(base16, L)]
                        av = (alb[pl.ds(ib * GBB + base16, L)]
                              / plsc.load_gather(d_v, [dv]))

                        def edge(g, _):
                            gg = base16 + g
                            bc = _dyn_gather16(
                                av, jnp.full((L,), g, jnp.int32))
                            for j in range(EMB // L):
                                sbuf[gg, pl.ds(j * L, L)] = (
                                    sbuf[gg, pl.ds(j * L, L)] * bc)
                            return 0
                        lax.fori_loop(0, L, edge, 0)
                        return 0
                    lax.fori_loop(0, GBB // L, subgrp, 0)

                    hs[ps] = pltpu.async_copy(
                        sbuf, sh_acc.at[idx_d2.at[ib]], SS[ps], add=True)
                    if ib < NB - 2:
                        pn = (ib + 2) & 3
                        if hs[pn] is not None:
                            hs[pn].wait()
                        hg[pn] = issue_g(ib + 2, pn)
                for ps in range(4):
                    hs[ps].wait()
                return 0
            lax.fori_loop(0, TILE_B // BLK_B, block, 0)
            plsc.subcore_barrier()

            for q in range(nrows // 40):
                pltpu.sync_copy(sh_acc.at[pl.ds(r0 + q * 40, 40)],
                                sb0.at[pl.ds(0, 40)])
                pltpu.sync_copy(sb0.at[pl.ds(0, 40)],
                                obl.at[pl.ds(r0 + q * 40, 40)])

        @pl.when(cid == 0)
        def _():
            chunk_pass(t0_hbm, o0_hbm, 0)
            chunk_pass(t1_hbm, o1_hbm, 1)

        @pl.when(cid == 1)
        def _():
            chunk_pass(t2_hbm, o2_hbm, 2)
            chunk_pass(t3_hbm, o3_hbm, 3)

    return k(src2, dst2, ex, den, t0, t1, t2, t3)


def _k15_body(dp_ref, out_ref):
    out_ref[...] = jnp.sum(dp_ref[...], axis=0) + 1e-16


def _k15(denp):
    return pl.pallas_call(
        _k15_body,
        out_shape=jax.ShapeDtypeStruct((HEADS, NP), jnp.float32),
    )(denp.reshape(NC * NS, HEADS, NP))


def _k2a_body(c0, c1, c2, c3, bias, stats):
    i = pl.program_id(0)
    y = jnp.concatenate([c0[...], c1[...], c2[...], c3[...]], axis=1) + bias[...]
    blk = jnp.concatenate([jnp.sum(y, axis=0, keepdims=True),
                           jnp.sum(y * y, axis=0, keepdims=True)], axis=0)

    @pl.when(i == 0)
    def _():
        stats[...] = blk

    @pl.when(i > 0)
    def _():
        stats[...] = stats[...] + blk


def _k2b_body(c0, c1, c2, c3, bias, stats1, g1, b1, wlin, w2, b2_, w3, b3_,
              z_ref, stats2):
    i = pl.program_id(0)
    y = jnp.concatenate([c0[...], c1[...], c2[...], c3[...]], axis=1) + bias[...]
    mean = stats1[0:1, :] * (1.0 / N)
    var = stats1[1:2, :] * (1.0 / N) - mean * mean
    yn = (y - mean) * lax.rsqrt(var + EPS) * g1[...] + b1[...]
    x1 = jnp.dot(yn, wlin[...], preferred_element_type=jnp.float32)
    h = jnp.maximum(jnp.dot(x1, w2[...], preferred_element_type=jnp.float32)
                    + b2_[...], 0.0)
    hh = jnp.dot(h, w3[...], preferred_element_type=jnp.float32) + b3_[...]
    z = x1 + hh
    z_ref[...] = z
    blk = jnp.concatenate([jnp.sum(z, axis=0, keepdims=True),
                           jnp.sum(z * z, axis=0, keepdims=True)], axis=0)

    @pl.when(i == 0)
    def _():
        stats2[...] = blk

    @pl.when(i > 0)
    def _():
        stats2[...] = stats2[...] + blk


def _k2c_body(z, stats2, g2, b2_, out):
    mean = stats2[0:1, :] * (1.0 / N)
    var = stats2[1:2, :] * (1.0 / N) - mean * mean
    out[...] = (z[...] - mean) * lax.rsqrt(var + EPS) * g2[...] + b2_[...]


def kernel(node_attr, edge_index, Wl, bl, Wr, br, att, bias_gat, gamma1,
           beta1, W_lin, W2, b2, W3, b3, gamma2, beta2):
    x = jnp.zeros((NP, IN_CH), jnp.float32).at[:N].set(node_attr)
    loop = jnp.arange(N, dtype=jnp.int32)
    pad = jnp.full((EP - E - N,), N, jnp.int32)
    src = jnp.concatenate([edge_index[0], loop, pad])
    dst = jnp.concatenate([edge_index[1], loop, pad])
    att_flat = att.reshape(HC)

    (xl, xr, xlb, xrb, t0, t1, t2, t3) = _k1(
        x, Wl, bl.reshape(1, HC), Wr, br.reshape(1, HC))
    ex, denp = _pass_a(xlb, xrb, src, dst, att_flat)

    denm = _k15(denp).reshape(HEADS * NP)
    o0, o1, o2, o3 = _pass_b(src.reshape(EP // GBB, GBB),
                             dst.reshape(EP // GBB, GBB), ex, denm,
                             t0, t1, t2, t3)

    rows = 1000
    stats1 = pl.pallas_call(
        _k2a_body,
        grid=(10,),
        in_specs=[pl.BlockSpec((rows, EMB), lambda i: (i, 0))] * 4
        + [pl.BlockSpec((1, HC), lambda i: (0, 0))],
        out_specs=pl.BlockSpec((2, HC), lambda i: (0, 0)),
        out_shape=jax.ShapeDtypeStruct((2, HC), jnp.float32),
    )(o0, o1, o2, o3, bias_gat.reshape(1, HC))

    z, stats2 = pl.pallas_call(
        _k2b_body,
        grid=(10,),
        in_specs=[pl.BlockSpec((rows, EMB), lambda i: (i, 0))] * 4
        + [pl.BlockSpec((1, HC), lambda i: (0, 0)),
           pl.BlockSpec((2, HC), lambda i: (0, 0)),
           pl.BlockSpec((1, HC), lambda i: (0, 0)),
           pl.BlockSpec((1, HC), lambda i: (0, 0)),
           pl.BlockSpec((HC, EMB), lambda i: (0, 0)),
           pl.BlockSpec((EMB, FF), lambda i: (0, 0)),
           pl.BlockSpec((1, FF), lambda i: (0, 0)),
           pl.BlockSpec((FF, EMB), lambda i: (0, 0)),
           pl.BlockSpec((1, EMB), lambda i: (0, 0))],
        out_specs=[pl.BlockSpec((rows, EMB), lambda i: (i, 0)),
                   pl.BlockSpec((2, EMB), lambda i: (0, 0))],
        out_shape=[jax.ShapeDtypeStruct((N, EMB), jnp.float32),
                   jax.ShapeDtypeStruct((2, EMB), jnp.float32)],
    )(o0, o1, o2, o3, bias_gat.reshape(1, HC), stats1, gamma1.reshape(1, HC),
      beta1.reshape(1, HC), W_lin, W2, b2.reshape(1, FF), W3,
      b3.reshape(1, EMB))

    out = pl.pallas_call(
        _k2c_body,
        grid=(10,),
        in_specs=[pl.BlockSpec((rows, EMB), lambda i: (i, 0)),
                  pl.BlockSpec((2, EMB), lambda i: (0, 0)),
                  pl.BlockSpec((1, EMB), lambda i: (0, 0)),
                  pl.BlockSpec((1, EMB), lambda i: (0, 0))],
        out_specs=pl.BlockSpec((rows, EMB), lambda i: (i, 0)),
        out_shape=jax.ShapeDtypeStruct((N, EMB), jnp.float32),
    )(z, stats2, gamma2.reshape(1, EMB), beta2.reshape(1, EMB))
    return out


# speedup vs baseline: 1.4676x; 1.0045x over previous
import functools

import jax
import jax.numpy as jnp
from jax import lax
from jax.experimental import pallas as pl
from jax.experimental.pallas import tpu as pltpu
from jax.experimental.pallas import tpu_sc as plsc

N = 10000
IN_CH = 256
EMB = 128
HEADS = 4
HC = HEADS * EMB
FF = 512
NEG = 0.2
EPS = 1e-5
E = 160000

NP = 10240
EP = 172032
NC, NS, L = 2, 16, 16
TILE_A = EP // (NC * NS)
TILE_B = EP // NS
GA = 128
GS = 64
GB = 128
NBA = TILE_A // GA
NBB = TILE_B // GB
DEN_W = NP * 4
DMR, DMC = DEN_W // 128, 128


def _dyn_gather16(v, idx):
    dnums = lax.GatherDimensionNumbers(
        offset_dims=(), collapsed_slice_dims=(0,), start_index_map=(0,))
    return lax.gather(v, idx[:, None], dnums, slice_sizes=(1,),
                      mode=lax.GatherScatterMode.PROMISE_IN_BOUNDS)


def _k1_body(x_ref, wl_ref, bl_ref, wr_ref, br_ref,
             xl_ref, xr_ref, xlb_ref, xrb_ref,
             c0_ref, c1_ref, c2_ref, c3_ref):
    x = x_ref[...]
    xl = jnp.dot(x, wl_ref[...], preferred_element_type=jnp.float32) + bl_ref[...]
    xr = jnp.dot(x, wr_ref[...], preferred_element_type=jnp.float32) + br_ref[...]
    xl_ref[...] = xl
    xr_ref[...] = xr
    xli = lax.bitcast_convert_type(
        xl.astype(jnp.bfloat16).astype(jnp.float32), jnp.int32)
    xri = lax.bitcast_convert_type(
        xr.astype(jnp.bfloat16).astype(jnp.float32), jnp.int32)
    xlb_ref[...] = lax.bitwise_or(
        lax.shift_right_logical(xli[:, 0:HC // 2], 16),
        lax.bitwise_and(xli[:, HC // 2:HC], -65536))
    xrb_ref[...] = lax.bitwise_or(
        lax.shift_right_logical(xri[:, 0:HC // 2], 16),
        lax.bitwise_and(xri[:, HC // 2:HC], -65536))
    c0_ref[...] = xl[:, 0:128]
    c1_ref[...] = xl[:, 128:256]
    c2_ref[...] = xl[:, 256:384]
    c3_ref[...] = xl[:, 384:512]


def _k1(x, Wl, bl2, Wr, br2):
    blk = NP // 8
    return pl.pallas_call(
        _k1_body,
        grid=(8,),
        in_specs=[
            pl.BlockSpec((blk, IN_CH), lambda i: (i, 0)),
            pl.BlockSpec((IN_CH, HC), lambda i: (0, 0)),
            pl.BlockSpec((1, HC), lambda i: (0, 0)),
            pl.BlockSpec((IN_CH, HC), lambda i: (0, 0)),
            pl.BlockSpec((1, HC), lambda i: (0, 0)),
        ],
        out_specs=[
            pl.BlockSpec((blk, HC), lambda i: (i, 0)),
            pl.BlockSpec((blk, HC), lambda i: (i, 0)),
            pl.BlockSpec((blk, HC // 2), lambda i: (i, 0)),
            pl.BlockSpec((blk, HC // 2), lambda i: (i, 0)),
            pl.BlockSpec((blk, EMB), lambda i: (i, 0)),
            pl.BlockSpec((blk, EMB), lambda i: (i, 0)),
            pl.BlockSpec((blk, EMB), lambda i: (i, 0)),
            pl.BlockSpec((blk, EMB), lambda i: (i, 0)),
        ],
        out_shape=[
            jax.ShapeDtypeStruct((NP, HC), jnp.float32),
            jax.ShapeDtypeStruct((NP, HC), jnp.float32),
            jax.ShapeDtypeStruct((NP, HC // 2), jnp.int32),
            jax.ShapeDtypeStruct((NP, HC // 2), jnp.int32),
            jax.ShapeDtypeStruct((NP, EMB), jnp.float32),
            jax.ShapeDtypeStruct((NP, EMB), jnp.float32),
            jax.ShapeDtypeStruct((NP, EMB), jnp.float32),
            jax.ShapeDtypeStruct((NP, EMB), jnp.float32),
        ],
    )(x, Wl, bl2, Wr, br2)


def _pass_a(xl, xr, src, dst, att_flat):
    mesh = plsc.VectorSubcoreMesh(core_axis_name="c", subcore_axis_name="s")
    HW = HC // 2

    @functools.partial(
        pl.kernel,
        mesh=mesh,
        compiler_params=pltpu.CompilerParams(needs_layout_passes=False),
        out_type=[
            jax.ShapeDtypeStruct((4, EP), jnp.float32),
            jax.ShapeDtypeStruct((NC * NS * DEN_W,), jnp.float32),
        ],
        scratch_types=[
            pltpu.VMEM((128,), jnp.int32),
            pltpu.VMEM((128,), jnp.int32),
            pltpu.VMEM((128,), jnp.int32),
            pltpu.VMEM((128,), jnp.int32),
            pltpu.VMEM((GS, HW), jnp.int32),
            pltpu.VMEM((GS, HW), jnp.int32),
            pltpu.VMEM((GS, HW), jnp.int32),
            pltpu.VMEM((GS, HW), jnp.int32),
            pltpu.VMEM((4, 256), jnp.float32),
            pltpu.VMEM((HC,), jnp.float32),
            pltpu.VMEM((DEN_W,), jnp.float32),
            pltpu.SemaphoreType.DMA,
            pltpu.SemaphoreType.DMA,
            pltpu.SemaphoreType.DMA,
            pltpu.SemaphoreType.DMA,
            pltpu.SemaphoreType.DMA,
            pltpu.SemaphoreType.DMA,
            pltpu.SemaphoreType.DMA,
            pltpu.SemaphoreType.DMA,
        ],
    )
    def k(xl_hbm, xr_hbm, src_hbm, dst_hbm, att_hbm, ex_hbm, den_hbm,
          ids0, ids1, idd0, idd1, rl0, rl1, rr0, rr1, ex_buf, att_v, den_v,
          sis0, sis1, sid0, sid1, sl0, sl1, sr0, sr1):
        cid = lax.axis_index("c")
        sid = lax.axis_index("s")
        tid = sid * NC + cid
        ii = lax.iota(jnp.int32, L)
        zv = jnp.zeros((L,), jnp.float32)
        IDS, IDD = (ids0, ids1), (idd0, idd1)
        SIS, SID = (sis0, sis1), (sid0, sid1)
        RL, RR = (rl0, rl1), (rr0, rr1)
        SL, SR = (sl0, sl1), (sr0, sr1)

        def zb(i, _):
            den_v[pl.ds(lax.mul(i, L), L)] = zv
            return 0
        lax.fori_loop(0, DEN_W // L, zb, 0)

        pltpu.sync_copy(att_hbm, att_v)
        att_sl = [att_v[pl.ds(w * L, L)] for w in range(HC // L)]

        ebase = lax.mul(tid, TILE_A)

        def issue_idx(jq, p):
            off = jnp.minimum(ebase + jq * 128, EP - 128)
            pltpu.async_copy(src_hbm.at[pl.ds(off, 128)], IDS[p], SIS[p])
            pltpu.async_copy(dst_hbm.at[pl.ds(off, 128)], IDD[p], SID[p])

        def wait_idx(p):
            pltpu.make_async_copy(
                src_hbm.at[pl.ds(0, 128)], IDS[p], SIS[p]).wait()
            pltpu.make_async_copy(
                dst_hbm.at[pl.ds(0, 128)], IDD[p], SID[p]).wait()

        def issue_g(half, b, pidx):
            pltpu.async_copy(
                xl_hbm.at[IDS[pidx].at[pl.ds(half * GS, GS)]], RL[b], SL[b])
            pltpu.async_copy(
                xr_hbm.at[IDD[pidx].at[pl.ds(half * GS, GS)]], RR[b], SR[b])

        def wait_g(b):
            pltpu.make_async_copy(
                xl_hbm.at[pl.ds(0, GS)], RL[b], SL[b]).wait()
            pltpu.make_async_copy(
                xr_hbm.at[pl.ds(0, GS)], RR[b], SR[b]).wait()

        def compute(half, b, pidx, exoff):
            rl, rr = RL[b], RR[b]
            idd = IDD[pidx]

            def subgrp(sg, _):
                colq = half * GS + sg * L
                col = exoff + colq

                def edge(g, lv):
                    gg = sg * L + g
                    sel = ii == g
                    accs = [jnp.zeros((L,), jnp.float32)
                            for _ in range(HEADS)]
                    for w in range(16):
                        wl = rl[gg, pl.ds(w * L, L)]
                        wr = rr[gg, pl.ds(w * L, L)]
                        le = plsc.bitcast(
                            lax.shift_left(wl, 16), jnp.float32)
                        he = plsc.bitcast(
                            lax.bitwise_and(wl, -65536), jnp.float32)
                        re_ = plsc.bitcast(
                            lax.shift_left(wr, 16), jnp.float32)
                        hr = plsc.bitcast(
                            lax.bitwise_and(wr, -65536), jnp.float32)
                        tl = le + re_
                        tl = jnp.maximum(tl, NEG * tl)
                        accs[w // 8] = accs[w // 8] + tl * att_sl[w]
                        th = he + hr
                        th = jnp.maximum(th, NEG * th)
                        accs[2 + w // 8] = (accs[2 + w // 8]
                                            + th * att_sl[16 + w])
                    for h in range(HEADS):
                        red = accs[h]
                        for st in (8, 4, 2, 1):
                            red = red + _dyn_gather16(
                                red, lax.bitwise_xor(ii, st))
                        lv = (lv[:h] + (jnp.where(sel, red, lv[h]),)
                              + lv[h + 1:])
                    return lv
                lv = lax.fori_loop(0, L, edge, (zv, zv, zv, zv))
                dv = idd[pl.ds(colq, L)]
                for h in range(HEADS):
                    ev = jnp.exp(lv[h])
                    ex_buf[h, pl.ds(col, L)] = ev
                    plsc.addupdate_scatter(den_v, [dv + h * NP], ev)
                return 0
            lax.fori_loop(0, GS // L, subgrp, 0)

        issue_idx(0, 0)
        issue_idx(1, 1)
        wait_idx(0)
        issue_g(0, 0, 0)
        issue_g(1, 1, 0)

        def pair(m, _):
            a2 = m * 2
            wait_g(0)
            compute(0, 0, 0, 0)
            wait_idx(1)
            issue_g(0, 0, 1)
            wait_g(1)
            compute(1, 1, 0, 0)
            issue_g(1, 1, 1)
            issue_idx(a2 + 2, 0)
            wait_g(0)
            compute(0, 0, 1, 128)
            wait_idx(0)
            issue_g(0, 0, 0)
            wait_g(1)
            compute(1, 1, 1, 128)
            issue_g(1, 1, 0)
            issue_idx(a2 + 3, 1)
            eb = ebase + m * 256
            for h in range(HEADS):
                pltpu.sync_copy(ex_buf.at[h],
                                ex_hbm.at[h].at[pl.ds(eb, 256)])
            return 0
        lax.fori_loop(0, TILE_A // 256, pair, 0)
        wait_g(0)
        wait_g(1)
        wait_idx(1)

        pltpu.sync_copy(den_v,
                        den_hbm.at[pl.ds(lax.mul(tid, DEN_W), DEN_W)])

    return k(xl, xr, src, dst, att_flat)


GBB = 48
BLK_B = 1536


def _pass_b(src2, dst2, ex, den, t0, t1, t2, t3):
    mesh = plsc.VectorSubcoreMesh(core_axis_name="c", subcore_axis_name="s")
    CW = EMB // 2

    @functools.partial(
        pl.kernel,
        mesh=mesh,
        compiler_params=pltpu.CompilerParams(needs_layout_passes=False),
        out_type=[jax.ShapeDtypeStruct((NP, EMB), jnp.float32)
                  for _ in range(4)],
        scratch_types=[
            pltpu.VMEM((BLK_B // GBB, GBB), jnp.int32),
            pltpu.VMEM((BLK_B // GBB, GBB), jnp.int32),
            pltpu.VMEM((BLK_B,), jnp.float32),
            pltpu.VMEM((NP,), jnp.float32),
            pltpu.VMEM((GBB, EMB), jnp.float32),
            pltpu.VMEM((GBB, EMB), jnp.float32),
            pltpu.VMEM((GBB, EMB), jnp.float32),
            pltpu.VMEM((GBB, EMB), jnp.float32),
            pltpu.VMEM_SHARED((NP, EMB), jnp.float32),
            pltpu.SemaphoreType.DMA,
            pltpu.SemaphoreType.DMA,
            pltpu.SemaphoreType.DMA,
            pltpu.SemaphoreType.DMA,
            pltpu.SemaphoreType.DMA,
            pltpu.SemaphoreType.DMA,
            pltpu.SemaphoreType.DMA,
            pltpu.SemaphoreType.DMA,
        ],
    )
    def k(src2_hbm, dst2_hbm, ex_hbm, den_hbm, t0_hbm, t1_hbm, t2_hbm,
          t3_hbm, o0_hbm, o1_hbm, o2_hbm, o3_hbm,
          idx_s2, idx_d2, alb, d_v, sb0, sb1, sb2, sb3, sh_acc,
          sg0, sg1, sg2, sg3, ss0, ss1, ss2, ss3):
        cid = lax.axis_index("c")
        sid = lax.axis_index("s")
        ii = lax.iota(jnp.int32, L)
        zv = jnp.zeros((L,), jnp.float32)
        SB = (sb0, sb1, sb2, sb3)
        SG = (sg0, sg1, sg2, sg3)
        SS = (ss0, ss1, ss2, ss3)
        NB = BLK_B // GBB

        nrows = NP // NS
        r0 = lax.mul(sid, nrows)
        ebase = lax.mul(sid, TILE_B)
        rbase = lax.mul(sid, TILE_B // GBB)

        def chunk_pass(tbl, obl, c):
            pltpu.sync_copy(den_hbm.at[pl.ds(c * NP, NP)], d_v)
            def zr(i, _):
                sb0[lax.shift_right_logical(i, 3),
                    pl.ds(lax.mul(lax.bitwise_and(i, 7), L), L)] = zv
                return 0
            lax.fori_loop(0, GBB * (EMB // L), zr, 0)
            for q in range(nrows // 40):
                pltpu.sync_copy(sb0.at[pl.ds(0, 40)],
                                sh_acc.at[pl.ds(r0 + q * 40, 40)])
            plsc.subcore_barrier()

            def issue_g(ib, p):
                return pltpu.async_copy(
                    tbl.at[idx_s2.at[ib]], SB[p], SG[p])

            def block(b, _):
                eb = ebase + b * BLK_B
                rr = rbase + b * NB
                pltpu.sync_copy(src2_hbm.at[pl.ds(rr, NB)], idx_s2)
                pltpu.sync_copy(dst2_hbm.at[pl.ds(rr, NB)], idx_d2)
                pltpu.sync_copy(ex_hbm.at[c].at[pl.ds(eb, BLK_B)], alb)
                hg = [issue_g(0, 0), issue_g(1, 1), None, None]
                hs = [None, None, None, None]
                for ib in range(NB):
                    ps = ib & 3
                    hg[ps].wait()
                    sbuf = SB[ps]

                    def subgrp(sg_, _):
                        base16 = lax.mul(sg_, L)
                        dv = idx_d2[ib, pl.ds(base16, L)]
                        av = (alb[pl.ds(ib * GBB + base16, L)]
                              / plsc.load_gather(d_v, [dv]))

                        def edge(g, _):
                            gg = base16 + g
                            bc = _dyn_gather16(
                                av, jnp.full((L,), g, jnp.int32))
                            for j in range(EMB // L):
                                sbuf[gg, pl.ds(j * L, L)] = (
                                    sbuf[gg, pl.ds(j * L, L)] * bc)
                            return 0
                        lax.fori_loop(0, L, edge, 0)
                        return 0
                    lax.fori_loop(0, GBB // L, subgrp, 0)

                    hs[ps] = pltpu.async_copy(
                        sbuf, sh_acc.at[idx_d2.at[ib]], SS[ps], add=True)
                    if ib < NB - 2:
                        pn = (ib + 2) & 3
                        if hs[pn] is not None:
                            hs[pn].wait()
                        hg[pn] = issue_g(ib + 2, pn)
                for ps in range(4):
                    hs[ps].wait()
                return 0
            lax.fori_loop(0, TILE_B // BLK_B, block, 0)
            plsc.subcore_barrier()

            for q in range(nrows // 40):
                pltpu.sync_copy(sh_acc.at[pl.ds(r0 + q * 40, 40)],
                                sb0.at[pl.ds(0, 40)])
                pltpu.sync_copy(sb0.at[pl.ds(0, 40)],
                                obl.at[pl.ds(r0 + q * 40, 40)])

        @pl.when(cid == 0)
        def _():
            chunk_pass(t0_hbm, o0_hbm, 0)
            chunk_pass(t1_hbm, o1_hbm, 1)

        @pl.when(cid == 1)
        def _():
            chunk_pass(t2_hbm, o2_hbm, 2)
            chunk_pass(t3_hbm, o3_hbm, 3)

    return k(src2, dst2, ex, den, t0, t1, t2, t3)


def _k15_body(dp_ref, out_ref):
    out_ref[...] = jnp.sum(dp_ref[...], axis=0) + 1e-16


def _k15(denp):
    return pl.pallas_call(
        _k15_body,
        out_shape=jax.ShapeDtypeStruct((HEADS, NP), jnp.float32),
    )(denp.reshape(NC * NS, HEADS, NP))


def _k2_body(c0, c1, c2, c3, bias, g1, b1, wlin, w2, b2_, w3, b3_, g2, be2,
             out, stats1, stats2, z_scr):
    i = pl.program_id(0)
    b = lax.rem(i, 10)

    @pl.when(i < 10)
    def _():
        y = jnp.concatenate([c0[...], c1[...], c2[...], c3[...]], axis=1)
        y = y + bias[...]
        blk = jnp.concatenate([jnp.sum(y, axis=0, keepdims=True),
                               jnp.sum(y * y, axis=0, keepdims=True)], axis=0)

        @pl.when(i == 0)
        def _():
            stats1[...] = blk

        @pl.when(i > 0)
        def _():
            stats1[...] = stats1[...] + blk

    @pl.when(jnp.logical_and(i >= 10, i < 20))
    def _():
        y = jnp.concatenate([c0[...], c1[...], c2[...], c3[...]], axis=1)
        y = y + bias[...]
        mean = stats1[0:1, :] * (1.0 / N)
        var = stats1[1:2, :] * (1.0 / N) - mean * mean
        yn = (y - mean) * lax.rsqrt(var + EPS) * g1[...] + b1[...]
        x1 = jnp.dot(yn, wlin[...], preferred_element_type=jnp.float32)
        h = jnp.maximum(
            jnp.dot(x1, w2[...], preferred_element_type=jnp.float32)
            + b2_[...], 0.0)
        hh = jnp.dot(h, w3[...], preferred_element_type=jnp.float32) + b3_[...]
        z = x1 + hh
        z_scr[pl.ds(b * 1000, 1000), :] = z
        blk2 = jnp.concatenate([jnp.sum(z, axis=0, keepdims=True),
                                jnp.sum(z * z, axis=0, keepdims=True)], axis=0)

        @pl.when(i == 10)
        def _():
            stats2[...] = blk2

        @pl.when(i > 10)
        def _():
            stats2[...] = stats2[...] + blk2

    @pl.when(i >= 20)
    def _():
        z = z_scr[pl.ds(b * 1000, 1000), :]
        mean = stats2[0:1, :] * (1.0 / N)
        var = stats2[1:2, :] * (1.0 / N) - mean * mean
        out[...] = (z - mean) * lax.rsqrt(var + EPS) * g2[...] + be2[...]


def kernel(node_attr, edge_index, Wl, bl, Wr, br, att, bias_gat, gamma1,
           beta1, W_lin, W2, b2, W3, b3, gamma2, beta2):
    x = jnp.zeros((NP, IN_CH), jnp.float32).at[:N].set(node_attr)
    loop = jnp.arange(N, dtype=jnp.int32)
    pad = jnp.full((EP - E - N,), N, jnp.int32)
    src = jnp.concatenate([edge_index[0], loop, pad])
    dst = jnp.concatenate([edge_index[1], loop, pad])
    att_flat = att.reshape(HC)

    (xl, xr, xlb, xrb, t0, t1, t2, t3) = _k1(
        x, Wl, bl.reshape(1, HC), Wr, br.reshape(1, HC))
    ex, denp = _pass_a(xlb, xrb, src, dst, att_flat)
    denm = _k15(denp).reshape(HEADS * NP)
    o0, o1, o2, o3 = _pass_b(src.reshape(EP // GBB, GBB),
                             dst.reshape(EP // GBB, GBB), ex, denm,
                             t0, t1, t2, t3)

    rows = 1000
    out = pl.pallas_call(
        _k2_body,
        grid=(30,),
        in_specs=[pl.BlockSpec((rows, EMB), lambda i: (lax.rem(i, 10), 0))] * 4
        + [pl.BlockSpec((1, HC), lambda i: (0, 0)),
           pl.BlockSpec((1, HC), lambda i: (0, 0)),
           pl.BlockSpec((1, HC), lambda i: (0, 0)),
           pl.BlockSpec((HC, EMB), lambda i: (0, 0)),
           pl.BlockSpec((EMB, FF), lambda i: (0, 0)),
           pl.BlockSpec((1, FF), lambda i: (0, 0)),
           pl.BlockSpec((FF, EMB), lambda i: (0, 0)),
           pl.BlockSpec((1, EMB), lambda i: (0, 0)),
           pl.BlockSpec((1, EMB), lambda i: (0, 0)),
           pl.BlockSpec((1, EMB), lambda i: (0, 0))],
        out_specs=pl.BlockSpec((rows, EMB), lambda i: (lax.rem(i, 10), 0)),
        out_shape=jax.ShapeDtypeStruct((N, EMB), jnp.float32),
        scratch_shapes=[
            pltpu.VMEM((2, HC), jnp.float32),
            pltpu.VMEM((2, EMB), jnp.float32),
            pltpu.VMEM((N, EMB), jnp.float32),
        ],
    )(o0, o1, o2, o3, bias_gat.reshape(1, HC), gamma1.reshape(1, HC),
      beta1.reshape(1, HC), W_lin, W2, b2.reshape(1, FF), W3,
      b3.reshape(1, EMB), gamma2.reshape(1, EMB), beta2.reshape(1, EMB))
    return out


# speedup vs baseline: 1.5061x; 1.0262x over previous
import functools

import jax
import jax.numpy as jnp
from jax import lax
from jax.experimental import pallas as pl
from jax.experimental.pallas import tpu as pltpu
from jax.experimental.pallas import tpu_sc as plsc

N = 10000
IN_CH = 256
EMB = 128
HEADS = 4
HC = HEADS * EMB
FF = 512
NEG = 0.2
EPS = 1e-5
E = 160000

NP = 10240
EP = 172032
NC, NS, L = 2, 16, 16
TILE_A = EP // (NC * NS)
TILE_B = EP // NS
GA = 128
GS = 64
GB = 128
NBA = TILE_A // GA
NBB = TILE_B // GB
DEN_W = NP * 4
DMR, DMC = DEN_W // 128, 128


def _dyn_gather16(v, idx):
    dnums = lax.GatherDimensionNumbers(
        offset_dims=(), collapsed_slice_dims=(0,), start_index_map=(0,))
    return lax.gather(v, idx[:, None], dnums, slice_sizes=(1,),
                      mode=lax.GatherScatterMode.PROMISE_IN_BOUNDS)


def _k1_body(x_ref, wl_ref, bl_ref, wr_ref, br_ref,
             xl_ref, xr_ref, xlb_ref, xrb_ref,
             c0_ref, c1_ref, c2_ref, c3_ref):
    x = x_ref[...]
    xl = jnp.dot(x, wl_ref[...], preferred_element_type=jnp.float32) + bl_ref[...]
    xr = jnp.dot(x, wr_ref[...], preferred_element_type=jnp.float32) + br_ref[...]
    xl_ref[...] = xl
    xr_ref[...] = xr
    xli = lax.bitcast_convert_type(
        xl.astype(jnp.bfloat16).astype(jnp.float32), jnp.int32)
    xri = lax.bitcast_convert_type(
        xr.astype(jnp.bfloat16).astype(jnp.float32), jnp.int32)
    xlb_ref[...] = lax.bitwise_or(
        lax.shift_right_logical(xli[:, 0:HC // 2], 16),
        lax.bitwise_and(xli[:, HC // 2:HC], -65536))
    xrb_ref[...] = lax.bitwise_or(
        lax.shift_right_logical(xri[:, 0:HC // 2], 16),
        lax.bitwise_and(xri[:, HC // 2:HC], -65536))
    c0_ref[...] = xl[:, 0:128]
    c1_ref[...] = xl[:, 128:256]
    c2_ref[...] = xl[:, 256:384]
    c3_ref[...] = xl[:, 384:512]


def _k1(x, Wl, bl2, Wr, br2):
    blk = NP // 8
    return pl.pallas_call(
        _k1_body,
        grid=(8,),
        in_specs=[
            pl.BlockSpec((blk, IN_CH), lambda i: (i, 0)),
            pl.BlockSpec((IN_CH, HC), lambda i: (0, 0)),
            pl.BlockSpec((1, HC), lambda i: (0, 0)),
            pl.BlockSpec((IN_CH, HC), lambda i: (0, 0)),
            pl.BlockSpec((1, HC), lambda i: (0, 0)),
        ],
        out_specs=[
            pl.BlockSpec((blk, HC), lambda i: (i, 0)),
            pl.BlockSpec((blk, HC), lambda i: (i, 0)),
            pl.BlockSpec((blk, HC // 2), lambda i: (i, 0)),
            pl.BlockSpec((blk, HC // 2), lambda i: (i, 0)),
            pl.BlockSpec((blk, EMB), lambda i: (i, 0)),
            pl.BlockSpec((blk, EMB), lambda i: (i, 0)),
            pl.BlockSpec((blk, EMB), lambda i: (i, 0)),
            pl.BlockSpec((blk, EMB), lambda i: (i, 0)),
        ],
        out_shape=[
            jax.ShapeDtypeStruct((NP, HC), jnp.float32),
            jax.ShapeDtypeStruct((NP, HC), jnp.float32),
            jax.ShapeDtypeStruct((NP, HC // 2), jnp.int32),
            jax.ShapeDtypeStruct((NP, HC // 2), jnp.int32),
            jax.ShapeDtypeStruct((NP, EMB), jnp.float32),
            jax.ShapeDtypeStruct((NP, EMB), jnp.float32),
            jax.ShapeDtypeStruct((NP, EMB), jnp.float32),
            jax.ShapeDtypeStruct((NP, EMB), jnp.float32),
        ],
    )(x, Wl, bl2, Wr, br2)


def _pass_a(xl, xr, src, dst, att_flat):
    mesh = plsc.VectorSubcoreMesh(core_axis_name="c", subcore_axis_name="s")
    HW = HC // 2

    @functools.partial(
        pl.kernel,
        mesh=mesh,
        compiler_params=pltpu.CompilerParams(needs_layout_passes=False),
        out_type=[
            jax.ShapeDtypeStruct((4, EP), jnp.float32),
            jax.ShapeDtypeStruct((NC * NS * DEN_W,), jnp.float32),
        ],
        scratch_types=[
            pltpu.VMEM((128,), jnp.int32),
            pltpu.VMEM((128,), jnp.int32),
            pltpu.VMEM((128,), jnp.int32),
            pltpu.VMEM((128,), jnp.int32),
            pltpu.VMEM((GS, HW), jnp.int32),
            pltpu.VMEM((GS, HW), jnp.int32),
            pltpu.VMEM((GS, HW), jnp.int32),
            pltpu.VMEM((GS, HW), jnp.int32),
            pltpu.VMEM((4, 256), jnp.float32),
            pltpu.VMEM((HC,), jnp.float32),
            pltpu.VMEM((DEN_W,), jnp.float32),
            pltpu.SemaphoreType.DMA,
            pltpu.SemaphoreType.DMA,
            pltpu.SemaphoreType.DMA,
            pltpu.SemaphoreType.DMA,
            pltpu.SemaphoreType.DMA,
            pltpu.SemaphoreType.DMA,
            pltpu.SemaphoreType.DMA,
            pltpu.SemaphoreType.DMA,
        ],
    )
    def k(xl_hbm, xr_hbm, src_hbm, dst_hbm, att_hbm, ex_hbm, den_hbm,
          ids0, ids1, idd0, idd1, rl0, rl1, rr0, rr1, ex_buf, att_v, den_v,
          sis0, sis1, sid0, sid1, sl0, sl1, sr0, sr1):
        cid = lax.axis_index("c")
        sid = lax.axis_index("s")
        tid = sid * NC + cid
        ii = lax.iota(jnp.int32, L)
        zv = jnp.zeros((L,), jnp.float32)
        IDS, IDD = (ids0, ids1), (idd0, idd1)
        SIS, SID = (sis0, sis1), (sid0, sid1)
        RL, RR = (rl0, rl1), (rr0, rr1)
        SL, SR = (sl0, sl1), (sr0, sr1)

        def zb(i, _):
            den_v[pl.ds(lax.mul(i, L), L)] = zv
            return 0
        lax.fori_loop(0, DEN_W // L, zb, 0)

        pltpu.sync_copy(att_hbm, att_v)
        att_sl = [att_v[pl.ds(w * L, L)] for w in range(HC // L)]

        ebase = lax.mul(tid, TILE_A)

        def issue_idx(jq, p):
            off = jnp.minimum(ebase + jq * 128, EP - 128)
            pltpu.async_copy(src_hbm.at[pl.ds(off, 128)], IDS[p], SIS[p])
            pltpu.async_copy(dst_hbm.at[pl.ds(off, 128)], IDD[p], SID[p])

        def wait_idx(p):
            pltpu.make_async_copy(
                src_hbm.at[pl.ds(0, 128)], IDS[p], SIS[p]).wait()
            pltpu.make_async_copy(
                dst_hbm.at[pl.ds(0, 128)], IDD[p], SID[p]).wait()

        def issue_g(half, b, pidx):
            pltpu.async_copy(
                xl_hbm.at[IDS[pidx].at[pl.ds(half * GS, GS)]], RL[b], SL[b])
            pltpu.async_copy(
                xr_hbm.at[IDD[pidx].at[pl.ds(half * GS, GS)]], RR[b], SR[b])

        def wait_g(b):
            pltpu.make_async_copy(
                xl_hbm.at[pl.ds(0, GS)], RL[b], SL[b]).wait()
            pltpu.make_async_copy(
                xr_hbm.at[pl.ds(0, GS)], RR[b], SR[b]).wait()

        def compute(half, b, pidx, exoff):
            rl, rr = RL[b], RR[b]
            idd = IDD[pidx]

            def subgrp(sg, _):
                colq = half * GS + sg * L
                col = exoff + colq

                def edge(g, lv):
                    gg = sg * L + g
                    sel = ii == g
                    accs = [jnp.zeros((L,), jnp.float32)
                            for _ in range(HEADS)]
                    for w in range(16):
                        wl = rl[gg, pl.ds(w * L, L)]
                        wr = rr[gg, pl.ds(w * L, L)]
                        le = plsc.bitcast(
                            lax.shift_left(wl, 16), jnp.float32)
                        he = plsc.bitcast(
                            lax.bitwise_and(wl, -65536), jnp.float32)
                        re_ = plsc.bitcast(
                            lax.shift_left(wr, 16), jnp.float32)
                        hr = plsc.bitcast(
                            lax.bitwise_and(wr, -65536), jnp.float32)
                        tl = le + re_
                        tl = jnp.maximum(tl, NEG * tl)
                        accs[w // 8] = accs[w // 8] + tl * att_sl[w]
                        th = he + hr
                        th = jnp.maximum(th, NEG * th)
                        accs[2 + w // 8] = (accs[2 + w // 8]
                                            + th * att_sl[16 + w])
                    for h in range(HEADS):
                        red = accs[h]
                        for st in (8, 4, 2, 1):
                            red = red + _dyn_gather16(
                                red, lax.bitwise_xor(ii, st))
                        lv = (lv[:h] + (jnp.where(sel, red, lv[h]),)
                              + lv[h + 1:])
                    return lv
                lv = lax.fori_loop(0, L, edge, (zv, zv, zv, zv))
                dv = idd[pl.ds(colq, L)]
                for h in range(HEADS):
                    ev = jnp.exp(lv[h])
                    ex_buf[h, pl.ds(col, L)] = ev
                    plsc.addupdate_scatter(den_v, [dv + h * NP], ev)
                return 0
            lax.fori_loop(0, GS // L, subgrp, 0)

        issue_idx(0, 0)
        issue_idx(1, 1)
        wait_idx(0)
        issue_g(0, 0, 0)
        issue_g(1, 1, 0)

        def pair(m, _):
            a2 = m * 2
            wait_g(0)
            compute(0, 0, 0, 0)
            wait_idx(1)
            issue_g(0, 0, 1)
            wait_g(1)
            compute(1, 1, 0, 0)
            issue_g(1, 1, 1)
            issue_idx(a2 + 2, 0)
            wait_g(0)
            compute(0, 0, 1, 128)
            wait_idx(0)
            issue_g(0, 0, 0)
            wait_g(1)
            compute(1, 1, 1, 128)
            issue_g(1, 1, 0)
            issue_idx(a2 + 3, 1)
            eb = ebase + m * 256
            for h in range(HEADS):
                pltpu.sync_copy(ex_buf.at[h],
                                ex_hbm.at[h].at[pl.ds(eb, 256)])
            return 0
        lax.fori_loop(0, TILE_A // 256, pair, 0)
        wait_g(0)
        wait_g(1)
        wait_idx(1)

        pltpu.sync_copy(den_v,
                        den_hbm.at[pl.ds(lax.mul(tid, DEN_W), DEN_W)])

    return k(xl, xr, src, dst, att_flat)


GBB = 64
NPA = 10112
BLK_B = 1536


def _pass_b(src2, dst2, ex, den, t0, t1, t2, t3):
    mesh = plsc.VectorSubcoreMesh(core_axis_name="c", subcore_axis_name="s")
    CW = EMB // 2

    @functools.partial(
        pl.kernel,
        mesh=mesh,
        compiler_params=pltpu.CompilerParams(needs_layout_passes=False),
        out_type=[jax.ShapeDtypeStruct((NP, EMB), jnp.float32)
                  for _ in range(4)],
        scratch_types=[
            pltpu.VMEM((BLK_B // GBB, GBB), jnp.int32),
            pltpu.VMEM((BLK_B // GBB, GBB), jnp.int32),
            pltpu.VMEM((BLK_B // 4,), jnp.float32),
            pltpu.VMEM((NP,), jnp.float32),
            pltpu.VMEM((GBB, EMB), jnp.float32),
            pltpu.VMEM((GBB, EMB), jnp.float32),
            pltpu.VMEM((GBB, EMB), jnp.float32),
            pltpu.VMEM((GBB, EMB), jnp.float32),
            pltpu.VMEM_SHARED((NPA, EMB), jnp.float32),
            pltpu.SemaphoreType.DMA,
            pltpu.SemaphoreType.DMA,
            pltpu.SemaphoreType.DMA,
            pltpu.SemaphoreType.DMA,
            pltpu.SemaphoreType.DMA,
            pltpu.SemaphoreType.DMA,
            pltpu.SemaphoreType.DMA,
            pltpu.SemaphoreType.DMA,
        ],
    )
    def k(src2_hbm, dst2_hbm, ex_hbm, den_hbm, t0_hbm, t1_hbm, t2_hbm,
          t3_hbm, o0_hbm, o1_hbm, o2_hbm, o3_hbm,
          idx_s2, idx_d2, alb, d_v, sb0, sb1, sb2, sb3, sh_acc,
          sg0, sg1, sg2, sg3, ss0, ss1, ss2, ss3):
        cid = lax.axis_index("c")
        sid = lax.axis_index("s")
        ii = lax.iota(jnp.int32, L)
        zv = jnp.zeros((L,), jnp.float32)
        SB = (sb0, sb1, sb2, sb3)
        SG = (sg0, sg1, sg2, sg3)
        SS = (ss0, ss1, ss2, ss3)
        NB = BLK_B // GBB

        nrows = NPA // NS
        r0 = lax.mul(sid, nrows)
        ebase = lax.mul(sid, TILE_B)
        rbase = lax.mul(sid, TILE_B // GBB)

        def chunk_pass(tbl, obl, c):
            pltpu.sync_copy(den_hbm.at[pl.ds(c * NP, NP)], d_v)
            def zr(i, _):
                sb0[lax.shift_right_logical(i, 3),
                    pl.ds(lax.mul(lax.bitwise_and(i, 7), L), L)] = zv
                return 0
            lax.fori_loop(0, GBB * (EMB // L), zr, 0)
            for q in range(15):
                pltpu.sync_copy(sb0.at[pl.ds(0, 40)],
                                sh_acc.at[pl.ds(r0 + q * 40, 40)])
            pltpu.sync_copy(sb0.at[pl.ds(0, 32)],
                            sh_acc.at[pl.ds(r0 + 600, 32)])
            plsc.subcore_barrier()

            def issue_g(ib, p):
                return pltpu.async_copy(
                    tbl.at[idx_s2.at[ib]], SB[p], SG[p])

            def block(b, _):
                eb = ebase + b * BLK_B
                rr = rbase + b * NB
                pltpu.sync_copy(src2_hbm.at[pl.ds(rr, NB)], idx_s2)
                pltpu.sync_copy(dst2_hbm.at[pl.ds(rr, NB)], idx_d2)
                hg = [issue_g(0, 0), issue_g(1, 1), None, None]
                hs = [None, None, None, None]
                for ib in range(NB):
                    if ib % 6 == 0:
                        pltpu.sync_copy(
                            ex_hbm.at[c].at[pl.ds(eb + ib * GBB, BLK_B // 4)],
                            alb)
                    ps = ib & 3
                    hg[ps].wait()
                    sbuf = SB[ps]

                    def subgrp(sg_, _):
                        base16 = lax.mul(sg_, L)
                        dv = idx_d2[ib, pl.ds(base16, L)]
                        av = (alb[pl.ds((ib % 6) * GBB + base16, L)]
                              / plsc.load_gather(d_v, [dv]))

                        def edge(g, _):
                            gg = base16 + g
                            bc = _dyn_gather16(
                                av, jnp.full((L,), g, jnp.int32))
                            for j in range(EMB // L):
                                sbuf[gg, pl.ds(j * L, L)] = (
                                    sbuf[gg, pl.ds(j * L, L)] * bc)
                            return 0
                        lax.fori_loop(0, L, edge, 0)
                        return 0
                    lax.fori_loop(0, GBB // L, subgrp, 0)

                    hs[ps] = pltpu.async_copy(
                        sbuf, sh_acc.at[idx_d2.at[ib]], SS[ps], add=True)
                    if ib < NB - 2:
                        pn = (ib + 2) & 3
                        if hs[pn] is not None:
                            hs[pn].wait()
                        hg[pn] = issue_g(ib + 2, pn)
                for ps in range(4):
                    hs[ps].wait()
                return 0
            lax.fori_loop(0, TILE_B // BLK_B, block, 0)
            plsc.subcore_barrier()

            for q in range(15):
                pltpu.sync_copy(sh_acc.at[pl.ds(r0 + q * 40, 40)],
                                sb0.at[pl.ds(0, 40)])
                pltpu.sync_copy(sb0.at[pl.ds(0, 40)],
                                obl.at[pl.ds(r0 + q * 40, 40)])
            pltpu.sync_copy(sh_acc.at[pl.ds(r0 + 600, 32)],
                            sb0.at[pl.ds(0, 32)])
            pltpu.sync_copy(sb0.at[pl.ds(0, 32)],
                            obl.at[pl.ds(r0 + 600, 32)])

        @pl.when(cid == 0)
        def _():
            chunk_pass(t0_hbm, o0_hbm, 0)
            chunk_pass(t1_hbm, o1_hbm, 1)

        @pl.when(cid == 1)
        def _():
            chunk_pass(t2_hbm, o2_hbm, 2)
            chunk_pass(t3_hbm, o3_hbm, 3)

    return k(src2, dst2, ex, den, t0, t1, t2, t3)


def _k15_body(dp_ref, out_ref):
    out_ref[...] = jnp.sum(dp_ref[...], axis=0) + 1e-16


def _k15(denp):
    return pl.pallas_call(
        _k15_body,
        out_shape=jax.ShapeDtypeStruct((HEADS, NP), jnp.float32),
    )(denp.reshape(NC * NS, HEADS, NP))


def _k2_body(c0, c1, c2, c3, bias, g1, b1, wlin, w2, b2_, w3, b3_, g2, be2,
             out, stats1, stats2, z_scr):
    i = pl.program_id(0)
    b = lax.rem(i, 10)

    @pl.when(i < 10)
    def _():
        y = jnp.concatenate([c0[...], c1[...], c2[...], c3[...]], axis=1)
        y = y + bias[...]
        blk = jnp.concatenate([jnp.sum(y, axis=0, keepdims=True),
                               jnp.sum(y * y, axis=0, keepdims=True)], axis=0)

        @pl.when(i == 0)
        def _():
            stats1[...] = blk

        @pl.when(i > 0)
        def _():
            stats1[...] = stats1[...] + blk

    @pl.when(jnp.logical_and(i >= 10, i < 20))
    def _():
        y = jnp.concatenate([c0[...], c1[...], c2[...], c3[...]], axis=1)
        y = y + bias[...]
        mean = stats1[0:1, :] * (1.0 / N)
        var = stats1[1:2, :] * (1.0 / N) - mean * mean
        yn = (y - mean) * lax.rsqrt(var + EPS) * g1[...] + b1[...]
        x1 = jnp.dot(yn, wlin[...], preferred_element_type=jnp.float32)
        h = jnp.maximum(
            jnp.dot(x1, w2[...], preferred_element_type=jnp.float32)
            + b2_[...], 0.0)
        hh = jnp.dot(h, w3[...], preferred_element_type=jnp.float32) + b3_[...]
        z = x1 + hh
        z_scr[pl.ds(b * 1000, 1000), :] = z
        blk2 = jnp.concatenate([jnp.sum(z, axis=0, keepdims=True),
                                jnp.sum(z * z, axis=0, keepdims=True)], axis=0)

        @pl.when(i == 10)
        def _():
            stats2[...] = blk2

        @pl.when(i > 10)
        def _():
            stats2[...] = stats2[...] + blk2

    @pl.when(i >= 20)
    def _():
        z = z_scr[pl.ds(b * 1000, 1000), :]
        mean = stats2[0:1, :] * (1.0 / N)
        var = stats2[1:2, :] * (1.0 / N) - mean * mean
        out[...] = (z - mean) * lax.rsqrt(var + EPS) * g2[...] + be2[...]


def kernel(node_attr, edge_index, Wl, bl, Wr, br, att, bias_gat, gamma1,
           beta1, W_lin, W2, b2, W3, b3, gamma2, beta2):
    x = jnp.zeros((NP, IN_CH), jnp.float32).at[:N].set(node_attr)
    loop = jnp.arange(N, dtype=jnp.int32)
    pad = jnp.full((EP - E - N,), N, jnp.int32)
    src = jnp.concatenate([edge_index[0], loop, pad])
    dst = jnp.concatenate([edge_index[1], loop, pad])
    att_flat = att.reshape(HC)

    (xl, xr, xlb, xrb, t0, t1, t2, t3) = _k1(
        x, Wl, bl.reshape(1, HC), Wr, br.reshape(1, HC))
    ex, denp = _pass_a(xlb, xrb, src, dst, att_flat)
    denm = _k15(denp).reshape(HEADS * NP)
    o0, o1, o2, o3 = _pass_b(src.reshape(EP // GBB, GBB),
                             dst.reshape(EP // GBB, GBB), ex, denm,
                             t0, t1, t2, t3)

    rows = 1000
    out = pl.pallas_call(
        _k2_body,
        grid=(30,),
        in_specs=[pl.BlockSpec((rows, EMB), lambda i: (lax.rem(i, 10), 0))] * 4
        + [pl.BlockSpec((1, HC), lambda i: (0, 0)),
           pl.BlockSpec((1, HC), lambda i: (0, 0)),
           pl.BlockSpec((1, HC), lambda i: (0, 0)),
           pl.BlockSpec((HC, EMB), lambda i: (0, 0)),
           pl.BlockSpec((EMB, FF), lambda i: (0, 0)),
           pl.BlockSpec((1, FF), lambda i: (0, 0)),
           pl.BlockSpec((FF, EMB), lambda i: (0, 0)),
           pl.BlockSpec((1, EMB), lambda i: (0, 0)),
           pl.BlockSpec((1, EMB), lambda i: (0, 0)),
           pl.BlockSpec((1, EMB), lambda i: (0, 0))],
        out_specs=pl.BlockSpec((rows, EMB), lambda i: (lax.rem(i, 10), 0)),
        out_shape=jax.ShapeDtypeStruct((N, EMB), jnp.float32),
        scratch_shapes=[
            pltpu.VMEM((2, HC), jnp.float32),
            pltpu.VMEM((2, EMB), jnp.float32),
            pltpu.VMEM((N, EMB), jnp.float32),
        ],
    )(o0, o1, o2, o3, bias_gat.reshape(1, HC), gamma1.reshape(1, HC),
      beta1.reshape(1, HC), W_lin, W2, b2.reshape(1, FF), W3,
      b3.reshape(1, EMB), gamma2.reshape(1, EMB), beta2.reshape(1, EMB))
    return out
